# baseline (speedup 1.0000x reference)
import functools

import jax
import jax.numpy as jnp
import numpy as np
from jax import lax
from jax.experimental import pallas as pl
from jax.experimental.pallas import tpu as pltpu
from jax.experimental.pallas import tpu_sc as plsc

F32 = jnp.float32
BF16 = jnp.bfloat16
HIDDEN = 256
HEADS = 16
HEAD_DIM = HIDDEN // HEADS
NRG = 20
EFD = 4
RFEAT = NRG * 4
ER = EFD + RFEAT
N_NODES = 10000
N_EDGES = 160000

NB = 400
EB = 640
CH = 64
NCHUNK = N_EDGES // CH
NSC = 2
NTILE = 16
NW = NSC * NTILE
HALF = N_NODES // NSC
ACC_ROWS = HALF + 120
GW = 3 * HIDDEN
SW = 2 * HIDDEN
S1W = HIDDEN + HEADS
S2W = 4 * HEADS

_GS_STEP = 10.0 / (NRG - 1)
_GS_COEFF = -0.5 / (_GS_STEP * _GS_STEP)


def _mesh():
    return plsc.VectorSubcoreMesh(core_axis_name="c", subcore_axis_name="s",
                                  num_cores=NSC, num_subcores=NTILE)


def _ln_relu(y, g, be):
    mu = jnp.mean(y, axis=-1, keepdims=True)
    yc = y - mu
    var = jnp.mean(yc * yc, axis=-1, keepdims=True)
    return jnp.maximum(yc * lax.rsqrt(var + 1e-5) * g + be, 0.0)


def _bc(shape):
    return pl.BlockSpec(shape, lambda i: (0,) * len(shape))


def _row(shape):
    return pl.BlockSpec(shape, lambda i: (i,) + (0,) * (len(shape) - 1))


def _node_pre_body(h_ref, wq0, bq0, gq, beq, wq1, bq1, wi, wj, b0c,
                   td_ref, ts_ref):
    h = h_ref[:]
    q = _ln_relu(h @ wq0[:] + bq0[:], gq[:], beq[:]) @ wq1[:] + bq1[:]
    td_ref[:] = jnp.concatenate([h @ wi[:] + b0c[:], q], axis=1).astype(BF16)
    ts_ref[:] = (h @ wj[:]).astype(BF16)


def _tc_node_pre(h, wq0, bq0, gq, beq, wq1, bq1, wi, wj, b0c):
    return pl.pallas_call(
        _node_pre_body,
        grid=(N_NODES // NB,),
        in_specs=[_row((NB, HIDDEN)), _bc((HIDDEN, HIDDEN)), _bc((1, HIDDEN)),
                  _bc((1, HIDDEN)), _bc((1, HIDDEN)), _bc((HIDDEN, HIDDEN)),
                  _bc((1, HIDDEN)), _bc((HIDDEN, SW)), _bc((HIDDEN, SW)),
                  _bc((1, SW))],
        out_specs=(_row((NB, GW)), _row((NB, SW))),
        out_shape=(jax.ShapeDtypeStruct((N_NODES, GW), BF16),
                   jax.ShapeDtypeStruct((N_NODES, SW), BF16)),
    )(h, wq0, bq0, gq, beq, wq1, bq1, wi, wj, b0c)


def _sc_gather_impl(td, ts, src, dst, xflat):
    with_rel = xflat is not None
    npair = ((NCHUNK + NW - 1) // NW + 1) // 2
    gw32 = GW // 2
    sw32 = SW // 2
    td = lax.bitcast_convert_type(td.reshape(N_NODES, gw32, 2), jnp.int32)
    ts = lax.bitcast_convert_type(ts.reshape(N_NODES, sw32, 2), jnp.int32)

    outs = [jax.ShapeDtypeStruct((N_EDGES, gw32), jnp.int32)]
    scratch = [pltpu.VMEM((CH,), jnp.int32), pltpu.VMEM((CH,), jnp.int32),
               pltpu.VMEM((CH, gw32), jnp.int32), pltpu.VMEM((CH, sw32), jnp.int32),
               pltpu.SemaphoreType.DMA, pltpu.SemaphoreType.DMA,
               pltpu.VMEM((CH,), jnp.int32), pltpu.VMEM((CH,), jnp.int32),
               pltpu.VMEM((CH, gw32), jnp.int32), pltpu.VMEM((CH, sw32), jnp.int32),
               pltpu.SemaphoreType.DMA, pltpu.SemaphoreType.DMA]
    if with_rel:
        outs.append(jax.ShapeDtypeStruct((N_EDGES * 4,), F32))
        scratch += [pltpu.VMEM((N_NODES * 3,), F32),
                    pltpu.VMEM((CH * 4,), F32)]

    def impl(td_h, ts_h, src_h, dst_h, x_h, g_out, r_out,
             b0, b1, x_v, rel_v):
        c = lax.axis_index("c")
        s = lax.axis_index("s")
        wid = s * NSC + c
        if with_rel:
            pltpu.sync_copy(x_h, x_v)
        lanes = lax.iota(jnp.int32, 16)

        def issue(buf, cid):
            idxs_v, idxd_v, rowd_v, rows_v, semd, sems = buf

            @pl.when(cid < NCHUNK)
            def _():
                e0 = cid * CH
                pltpu.sync_copy(src_h.at[pl.ds(e0, CH)], idxs_v)
                pltpu.sync_copy(dst_h.at[pl.ds(e0, CH)], idxd_v)
                pltpu.async_copy(td_h.at[idxd_v], rowd_v, semd)
                pltpu.async_copy(ts_h.at[idxs_v], rows_v, sems)

        def process(buf, cid):
            idxs_v, idxd_v, rowd_v, rows_v, semd, sems = buf

            @pl.when(cid < NCHUNK)
            def _():
                e0 = cid * CH
                pltpu.make_async_copy(td_h.at[idxd_v], rowd_v, semd).wait()
                pltpu.make_async_copy(ts_h.at[idxs_v], rows_v, sems).wait()

                def add_row(i, cy):
                    for j in range(sw32 // 16):
                        sl = pl.ds(16 * j, 16)
                        a = plsc.bitcast(rowd_v[i, sl], BF16)
                        b = plsc.bitcast(rows_v[i, sl], BF16)
                        rowd_v[i, sl] = plsc.bitcast(a + b, jnp.int32)
                    return cy

                lax.fori_loop(0, CH, add_row, 0)
                if with_rel:
                    for g_ in range(CH // 16):
                        sv = idxs_v[pl.ds(16 * g_, 16)] * 3
                        dv = idxd_v[pl.ds(16 * g_, 16)] * 3
                        flat = (lanes + 16 * g_) * 4
                        d2 = jnp.zeros((16,), F32)
                        for comp in range(3):
                            xs = plsc.load_gather(x_v, [sv + comp])
                            xd = plsc.load_gather(x_v, [dv + comp])
                            r = xd - xs
                            plsc.store_scatter(rel_v, [flat + comp], r)
                            d2 = d2 + r * r
                        plsc.store_scatter(rel_v, [flat + 3], d2)
                    pltpu.sync_copy(rel_v, r_out.at[pl.ds(e0 * 4, CH * 4)])
                pltpu.sync_copy(rowd_v, g_out.at[pl.ds(e0, CH)])

        issue(b0, wid)

        def pair_body(p, carry):
            k0 = 2 * p
            issue(b1, wid + NW * (k0 + 1))
            process(b0, wid + NW * k0)
            issue(b0, wid + NW * (k0 + 2))
            process(b1, wid + NW * (k0 + 1))
            return carry

        lax.fori_loop(0, npair, pair_body, 0)

    if with_rel:
        @functools.partial(
            pl.kernel,
            mesh=_mesh(),
            compiler_params=pltpu.CompilerParams(needs_layout_passes=False),
            out_type=tuple(outs),
            scratch_types=scratch,
        )
        def body(td_h, ts_h, src_h, dst_h, x_h, g_out, r_out,
                 i0s, i0d, r0d, r0s, s0d, s0s,
                 i1s, i1d, r1d, r1s, s1d, s1s, x_v, rel_v):
            impl(td_h, ts_h, src_h, dst_h, x_h, g_out, r_out,
                 (i0s, i0d, r0d, r0s, s0d, s0s),
                 (i1s, i1d, r1d, r1s, s1d, s1s), x_v, rel_v)

        g32, relflat = body(td, ts, src, dst, xflat)
        g = lax.bitcast_convert_type(g32, BF16).reshape(N_EDGES, GW)
        return g, relflat.reshape(N_EDGES, 4)
    else:
        @functools.partial(
            pl.kernel,
            mesh=_mesh(),
            compiler_params=pltpu.CompilerParams(needs_layout_passes=False),
            out_type=tuple(outs),
            scratch_types=scratch,
        )
        def body(td_h, ts_h, src_h, dst_h, g_out,
                 i0s, i0d, r0d, r0s, s0d, s0s,
                 i1s, i1d, r1d, r1s, s1d, s1s):
            impl(td_h, ts_h, src_h, dst_h, None, g_out, None,
                 (i0s, i0d, r0d, r0s, s0d, s0s),
                 (i1s, i1d, r1d, r1s, s1d, s1s), None, None)

        res = body(td, ts, src, dst)
        g32 = res[0] if isinstance(res, (tuple, list)) else res
        return lax.bitcast_convert_type(g32, BF16).reshape(N_EDGES, GW)


def _sc_gather_rel(td, ts, src, dst, x):
    return _sc_gather_impl(td, ts, src, dst, x.reshape(-1))


def _sc_gather(td, ts, src, dst):
    return _sc_gather_impl(td, ts, src, dst, None)


def _sc_scatter_add(data, dst, width):
    zeros = jnp.zeros((ACC_ROWS, width), F32)
    rz = ACC_ROWS // NTILE
    rpt = 320
    tail = HALF - (NTILE - 1) * rpt

    @functools.partial(
        pl.kernel,
        mesh=_mesh(),
        compiler_params=pltpu.CompilerParams(needs_layout_passes=False,
                                             use_tc_tiling_on_sc=False),
        out_type=jax.ShapeDtypeStruct((NSC, HALF, width), F32),
        scratch_types=[pltpu.VMEM((CH,), jnp.int32),
                       pltpu.VMEM((CH,), jnp.int32),
                       pltpu.VMEM((CH, width), F32),
                       pltpu.VMEM_SHARED((ACC_ROWS, width), F32),
                       pltpu.SemaphoreType.DMA],
    )
    def body(data_h, dst_h, z_h, out_h, idxd_v, idxl_v, data_v, acc_sh, sem):
        c = lax.axis_index("c")
        s = lax.axis_index("s")
        base = c * HALF
        pltpu.sync_copy(z_h.at[pl.ds(rz * s, rz)], acc_sh.at[pl.ds(rz * s, rz)])
        plsc.subcore_barrier()

        def chunk_body(k, carry):
            cid = s + NTILE * k

            @pl.when(cid < NCHUNK)
            def _():
                e0 = cid * CH
                pltpu.sync_copy(dst_h.at[pl.ds(e0, CH)], idxd_v)
                pltpu.sync_copy(data_h.at[pl.ds(e0, CH)], data_v)
                for g_ in range(CH // 16):
                    sl = pl.ds(16 * g_, 16)
                    loc = idxd_v[sl] - base
                    ok = (loc >= 0) & (loc < HALF)
                    idxl_v[sl] = jnp.where(ok, loc, HALF)
                pltpu.sync_copy(data_v, acc_sh.at[idxl_v], add=True)
            return carry

        lax.fori_loop(0, (NCHUNK + NTILE - 1) // NTILE, chunk_body, 0)
        plsc.subcore_barrier()

        @pl.when(s < NTILE - 1)
        def _():
            pltpu.sync_copy(acc_sh.at[pl.ds(rpt * s, rpt)],
                            out_h.at[c, pl.ds(rpt * s, rpt)])

        @pl.when(s == NTILE - 1)
        def _():
            pltpu.sync_copy(acc_sh.at[pl.ds(rpt * s, tail)],
                            out_h.at[c, pl.ds(rpt * s, tail)])

    return body(data, dst, zeros).reshape(N_NODES, width)


def _r_feat(ea, rel):
    d2 = rel[:, 3:4]
    dist = jnp.sqrt(d2)
    offs = lax.broadcasted_iota(jnp.int32, (1, NRG), 1).astype(F32) * _GS_STEP
    df = jnp.exp(_GS_COEFF * (dist - offs) ** 2)
    rf = jnp.concatenate([ea[:, a:a + 1] * df for a in range(EFD)], axis=1)
    return rf


def _edge1_body(g_ref, ea_ref, rel_ref, wer, ghk, behk, ghv, behv,
                w1hk, b1hk, w1hv, b1hv, eww, ewb, prep, psum, s1_ref):
    g = g_ref[:].astype(F32)
    ea = ea_ref[:]
    rel = rel_ref[:]
    rf = _r_feat(ea, rel)
    er = jnp.concatenate([ea, rf], axis=1)
    pre = er @ wer[:] + g[:, :SW]
    k = _ln_relu(pre[:, :HIDDEN], ghk[:], behk[:]) @ w1hk[:] + b1hk[:]
    v = _ln_relu(pre[:, HIDDEN:], ghv[:], behv[:]) @ w1hv[:] + b1hv[:]
    ew = jax.nn.sigmoid(rf @ eww[:] + ewb[:])
    v = v * ew
    logits = ((g[:, SW:] * k) @ psum[:]) * 0.25
    ex = jnp.exp(logits)
    s1_ref[:] = jnp.concatenate([(ex @ prep[:]) * v, ex], axis=1)


def _tc_edge1(g1, ea, rel, wer, ghk, behk, ghv, behv, w1hk, b1hk,
              w1hv, b1hv, eww, ewb, prep, psum):
    return pl.pallas_call(
        _edge1_body,
        grid=(N_EDGES // EB,),
        in_specs=[_row((EB, GW)), _row((EB, EFD)), _row((EB, 4)),
                  _bc((ER, SW)), _bc((1, HIDDEN)), _bc((1, HIDDEN)),
                  _bc((1, HIDDEN)), _bc((1, HIDDEN)),
                  _bc((HIDDEN, HIDDEN)), _bc((1, HIDDEN)),
                  _bc((HIDDEN, HIDDEN)), _bc((1, HIDDEN)),
                  _bc((RFEAT, 1)), _bc((1, 1)),
                  _bc((HEADS, HIDDEN)), _bc((HIDDEN, HEADS))],
        out_specs=_row((EB, S1W)),
        out_shape=jax.ShapeDtypeStruct((N_EDGES, S1W), F32),
    )(g1, ea, rel, wer, ghk, behk, ghv, behv, w1hk, b1hk, w1hv, b1hv,
      eww, ewb, prep, psum)


def _edge2_body(g_ref, ea_ref, rel_ref, wer, gxk, bexk, gxv, bexv,
                w1xk, b1xk, w1xv, b1xv, eww, ewb, psum, s2_ref):
    g = g_ref[:].astype(F32)
    ea = ea_ref[:]
    rel = rel_ref[:]
    rf = _r_feat(ea, rel)
    er = jnp.concatenate([ea, rf], axis=1)
    pre = er @ wer[:] + g[:, :SW]
    k = _ln_relu(pre[:, :HIDDEN], gxk[:], bexk[:]) @ w1xk[:] + b1xk[:]
    v = _ln_relu(pre[:, HIDDEN:], gxv[:], bexv[:]) @ w1xv[:] + b1xv[:]
    ew = jax.nn.sigmoid(rf @ eww[:] + ewb[:])
    v = v * ew
    logits = ((g[:, SW:] * k) @ psum[:]) * 0.25
    ex = jnp.exp(logits)
    w = ex * v
    s2_ref[:] = jnp.concatenate([w * rel[:, 0:1], w * rel[:, 1:2],
                                 w * rel[:, 2:3], ex], axis=1)


def _tc_edge2(g2, ea, rel, wer, gxk, bexk, gxv, bexv, w1xk, b1xk,
              w1xv, b1xv, eww, ewb, psum):
    return pl.pallas_call(
        _edge2_body,
        grid=(N_EDGES // EB,),
        in_specs=[_row((EB, GW)), _row((EB, EFD)), _row((EB, 4)),
                  _bc((ER, SW)), _bc((1, HIDDEN)), _bc((1, HIDDEN)),
                  _bc((1, HIDDEN)), _bc((1, HIDDEN)),
                  _bc((HIDDEN, HIDDEN)), _bc((1, HIDDEN)),
                  _bc((HIDDEN, HEADS)), _bc((1, HEADS)),
                  _bc((RFEAT, 1)), _bc((1, 1)), _bc((HIDDEN, HEADS))],
        out_specs=_row((EB, S2W)),
        out_shape=jax.ShapeDtypeStruct((N_EDGES, S2W), F32),
    )(g2, ea, rel, wer, gxk, bexk, gxv, bexv, w1xk, b1xk, w1xv, b1xv,
      eww, ewb, psum)


def _node_out_body(acc_ref, h_ref, w0no, b0no, gno, beno, w1no, b1no, prep,
                   wq0, bq0, gq, beq, wq1, bq1, wi, wj, b0c,
                   hout_ref, td_ref, ts_ref):
    a = acc_ref[:]
    h = h_ref[:]
    den = (a[:, HIDDEN:] @ prep[:]) + 1e-16
    attn = a[:, :HIDDEN] / den
    z = jnp.concatenate([attn, h], axis=1) @ w0no[:] + b0no[:]
    hout = _ln_relu(z, gno[:], beno[:]) @ w1no[:] + b1no[:] + h
    hout_ref[:] = hout
    q = _ln_relu(hout @ wq0[:] + bq0[:], gq[:], beq[:]) @ wq1[:] + bq1[:]
    td_ref[:] = jnp.concatenate([hout @ wi[:] + b0c[:], q], axis=1).astype(BF16)
    ts_ref[:] = (hout @ wj[:]).astype(BF16)


def _tc_node_out(acc1, h, w0no, b0no, gno, beno, w1no, b1no, prep,
                 wq0, bq0, gq, beq, wq1, bq1, wi, wj, b0c):
    return pl.pallas_call(
        _node_out_body,
        grid=(N_NODES // NB,),
        in_specs=[_row((NB, S1W)), _row((NB, HIDDEN)),
                  _bc((2 * HIDDEN, HIDDEN)), _bc((1, HIDDEN)),
                  _bc((1, HIDDEN)), _bc((1, HIDDEN)),
                  _bc((HIDDEN, HIDDEN)), _bc((1, HIDDEN)),
                  _bc((HEADS, HIDDEN)),
                  _bc((HIDDEN, HIDDEN)), _bc((1, HIDDEN)), _bc((1, HIDDEN)),
                  _bc((1, HIDDEN)), _bc((HIDDEN, HIDDEN)), _bc((1, HIDDEN)),
                  _bc((HIDDEN, SW)), _bc((HIDDEN, SW)), _bc((1, SW))],
        out_specs=(_row((NB, HIDDEN)), _row((NB, GW)), _row((NB, SW))),
        out_shape=(jax.ShapeDtypeStruct((N_NODES, HIDDEN), F32),
                   jax.ShapeDtypeStruct((N_NODES, GW), BF16),
                   jax.ShapeDtypeStruct((N_NODES, SW), BF16)),
    )(acc1, h, w0no, b0no, gno, beno, w1no, b1no, prep,
      wq0, bq0, gq, beq, wq1, bq1, wi, wj, b0c)


def _finalize_body(acc_ref, x_ref, m_ref, xo_ref):
    a = acc_ref[:]
    inv = 1.0 / (a[:, 3 * HEADS:] + 1e-16)
    one = jnp.ones((HEADS, 1), F32)
    scale = 1.0 / HEADS
    parts = [((a[:, c * HEADS:(c + 1) * HEADS] * inv) @ one) * scale
             for c in range(3)]
    delta = jnp.concatenate(parts, axis=1)
    xo_ref[:] = x_ref[:] + delta * m_ref[:]


def _tc_finalize(acc2, x, mask):
    return pl.pallas_call(
        _finalize_body,
        grid=(N_NODES // NB,),
        in_specs=[_row((NB, S2W)), _row((NB, 3)), _row((NB, 1))],
        out_specs=_row((NB, 3)),
        out_shape=jax.ShapeDtypeStruct((N_NODES, 3), F32),
    )(acc2, x, mask)


def _split_kv(w0):
    return w0[:ER], w0[ER:ER + HIDDEN], w0[ER + HIDDEN:]


def kernel(h, x, edge_attr, edge_index, mask_ligand, params):
    src = edge_index[0]
    dst = edge_index[1]
    p1 = params["x2h"]
    p2 = params["h2x"]

    er_hk, wi_hk, wj_hk = _split_kv(p1["hk"]["w0"])
    er_hv, wi_hv, wj_hv = _split_kv(p1["hv"]["w0"])
    wi1 = jnp.concatenate([wi_hk, wi_hv], axis=1)
    wj1 = jnp.concatenate([wj_hk, wj_hv], axis=1)
    wer1 = jnp.concatenate([er_hk, er_hv], axis=1)
    b01 = jnp.concatenate([p1["hk"]["b0"], p1["hv"]["b0"]])[None, :]

    er_xk, wi_xk, wj_xk = _split_kv(p2["xk"]["w0"])
    er_xv, wi_xv, wj_xv = _split_kv(p2["xv"]["w0"])
    wi2 = jnp.concatenate([wi_xk, wi_xv], axis=1)
    wj2 = jnp.concatenate([wj_xk, wj_xv], axis=1)
    wer2 = jnp.concatenate([er_xk, er_xv], axis=1)
    b02 = jnp.concatenate([p2["xk"]["b0"], p2["xv"]["b0"]])[None, :]

    prep = jnp.asarray(np.kron(np.eye(HEADS, dtype=np.float32),
                               np.ones((1, HEAD_DIM), np.float32)))
    psum = prep.T

    def r1(v):
        return v[None, :]

    hq = p1["hq"]
    td1, ts1 = _tc_node_pre(h, hq["w0"], r1(hq["b0"]), r1(hq["g"]),
                            r1(hq["be"]), hq["w1"], r1(hq["b1"]),
                            wi1, wj1, b01)

    g1, rel = _sc_gather_rel(td1, ts1, src, dst, x)

    s1 = _tc_edge1(g1, edge_attr, rel, wer1,
                   r1(p1["hk"]["g"]), r1(p1["hk"]["be"]),
                   r1(p1["hv"]["g"]), r1(p1["hv"]["be"]),
                   p1["hk"]["w1"], r1(p1["hk"]["b1"]),
                   p1["hv"]["w1"], r1(p1["hv"]["b1"]),
                   p1["ew_w"], p1["ew_b"][None, :], prep, psum)

    acc1 = _sc_scatter_add(s1, dst, S1W)

    no = p1["node_out"]
    xq = p2["xq"]
    h_out, td2, ts2 = _tc_node_out(
        acc1, h, no["w0"], r1(no["b0"]), r1(no["g"]), r1(no["be"]),
        no["w1"], r1(no["b1"]), prep,
        xq["w0"], r1(xq["b0"]), r1(xq["g"]), r1(xq["be"]),
        xq["w1"], r1(xq["b1"]), wi2, wj2, b02)

    g2 = _sc_gather(td2, ts2, src, dst)

    s2 = _tc_edge2(g2, edge_attr, rel, wer2,
                   r1(p2["xk"]["g"]), r1(p2["xk"]["be"]),
                   r1(p2["xv"]["g"]), r1(p2["xv"]["be"]),
                   p2["xk"]["w1"], r1(p2["xk"]["b1"]),
                   p2["xv"]["w1"], r1(p2["xv"]["b1"]),
                   p2["ew_w"], p2["ew_b"][None, :], psum)

    acc2 = _sc_scatter_add(s2, dst, S2W)

    x_out = _tc_finalize(acc2, x, mask_ligand[:, None])
    return h_out, x_out


# speedup vs baseline: 2.4248x; 2.4248x over previous
import functools

import jax
import jax.numpy as jnp
import numpy as np
from jax import lax
from jax.experimental import pallas as pl
from jax.experimental.pallas import tpu as pltpu
from jax.experimental.pallas import tpu_sc as plsc

F32 = jnp.float32
BF16 = jnp.bfloat16
HIDDEN = 256
HEADS = 16
HEAD_DIM = HIDDEN // HEADS
NRG = 20
EFD = 4
RFEAT = NRG * 4
ER = EFD + RFEAT
N_NODES = 10000
N_EDGES = 160000

NB = 400
EB = 640
CH = 64
NCHUNK = N_EDGES // CH
NSC = 2
NTILE = 16
NW = NSC * NTILE
HALF = N_NODES // NSC
ACC_ROWS = HALF + 120
GW = 3 * HIDDEN
SW = 2 * HIDDEN
GW2 = GW // 2
SW2 = SW // 2
S1W = HIDDEN + HEADS
S2W = 4 * HEADS

_GS_STEP = 10.0 / (NRG - 1)
_GS_COEFF = -0.5 / (_GS_STEP * _GS_STEP)


def _mesh():
    return plsc.VectorSubcoreMesh(core_axis_name="c", subcore_axis_name="s",
                                  num_cores=NSC, num_subcores=NTILE)


def _ln_relu(y, g, be):
    mu = jnp.mean(y, axis=-1, keepdims=True)
    yc = y - mu
    var = jnp.mean(yc * yc, axis=-1, keepdims=True)
    return jnp.maximum(yc * lax.rsqrt(var + 1e-5) * g + be, 0.0)


def _pack(a, b):
    au = lax.bitcast_convert_type(a.astype(BF16), jnp.uint16).astype(jnp.uint32)
    bu = lax.bitcast_convert_type(b.astype(BF16), jnp.uint16).astype(jnp.uint32)
    return lax.bitcast_convert_type(au | (bu << 16), jnp.int32)


def _unpack(g):
    gu = lax.bitcast_convert_type(g, jnp.uint32)
    lo = lax.bitcast_convert_type((gu & 0xFFFF).astype(jnp.uint16), BF16)
    hi = lax.bitcast_convert_type((gu >> 16).astype(jnp.uint16), BF16)
    return lo, hi


def _bc(shape):
    return pl.BlockSpec(shape, lambda i: (0,) * len(shape))


def _row(shape):
    return pl.BlockSpec(shape, lambda i: (i,) + (0,) * (len(shape) - 1))


def _pack_rows(u, q):
    return jnp.concatenate(
        [_pack(u[:, :SW2], u[:, SW2:]),
         _pack(q[:, :HIDDEN // 2], q[:, HIDDEN // 2:])], axis=1)


def _node_pre_body(h_ref, wq0, bq0, gq, beq, wq1, bq1, wi, wj, b0c,
                   td_ref, ts_ref):
    h = h_ref[:]
    q = _ln_relu(h @ wq0[:] + bq0[:], gq[:], beq[:]) @ wq1[:] + bq1[:]
    u = h @ wi[:] + b0c[:]
    s = h @ wj[:]
    td_ref[:] = _pack_rows(u, q)
    ts_ref[:] = _pack(s[:, :SW2], s[:, SW2:])


def _tc_node_pre(h, wq0, bq0, gq, beq, wq1, bq1, wi, wj, b0c):
    return pl.pallas_call(
        _node_pre_body,
        grid=(N_NODES // NB,),
        in_specs=[_row((NB, HIDDEN)), _bc((HIDDEN, HIDDEN)), _bc((1, HIDDEN)),
                  _bc((1, HIDDEN)), _bc((1, HIDDEN)), _bc((HIDDEN, HIDDEN)),
                  _bc((1, HIDDEN)), _bc((HIDDEN, SW)), _bc((HIDDEN, SW)),
                  _bc((1, SW))],
        out_specs=(_row((NB, GW2)), _row((NB, SW2))),
        out_shape=(jax.ShapeDtypeStruct((N_NODES, GW2), jnp.int32),
                   jax.ShapeDtypeStruct((N_NODES, SW2), jnp.int32)),
    )(h, wq0, bq0, gq, beq, wq1, bq1, wi, wj, b0c)


def _sc_gather_impl(td, ts, src, dst, xflat):
    with_rel = xflat is not None
    npair = ((NCHUNK + NW - 1) // NW + 1) // 2
    gw32 = GW2
    sw32 = SW2

    outs = [jax.ShapeDtypeStruct((N_EDGES, gw32), jnp.int32)]
    scratch = [pltpu.VMEM((CH,), jnp.int32), pltpu.VMEM((CH,), jnp.int32),
               pltpu.VMEM((CH, gw32), jnp.int32), pltpu.VMEM((CH, sw32), jnp.int32),
               pltpu.SemaphoreType.DMA, pltpu.SemaphoreType.DMA,
               pltpu.VMEM((CH,), jnp.int32), pltpu.VMEM((CH,), jnp.int32),
               pltpu.VMEM((CH, gw32), jnp.int32), pltpu.VMEM((CH, sw32), jnp.int32),
               pltpu.SemaphoreType.DMA, pltpu.SemaphoreType.DMA]
    if with_rel:
        outs.append(jax.ShapeDtypeStruct((N_EDGES * 4,), F32))
        scratch += [pltpu.VMEM((N_NODES * 3,), F32),
                    pltpu.VMEM((CH * 4,), F32)]

    def impl(td_h, ts_h, src_h, dst_h, x_h, g_out, r_out,
             b0, b1, x_v, rel_v):
        c = lax.axis_index("c")
        s = lax.axis_index("s")
        wid = s * NSC + c
        if with_rel:
            pltpu.sync_copy(x_h, x_v)
        lanes = lax.iota(jnp.int32, 16)

        def issue(buf, cid):
            idxs_v, idxd_v, rowd_v, rows_v, semd, sems = buf

            @pl.when(cid < NCHUNK)
            def _():
                e0 = cid * CH
                pltpu.sync_copy(src_h.at[pl.ds(e0, CH)], idxs_v)
                pltpu.sync_copy(dst_h.at[pl.ds(e0, CH)], idxd_v)
                pltpu.async_copy(td_h.at[idxd_v], rowd_v, semd)
                pltpu.async_copy(ts_h.at[idxs_v], rows_v, sems)

        def process(buf, cid):
            idxs_v, idxd_v, rowd_v, rows_v, semd, sems = buf

            @pl.when(cid < NCHUNK)
            def _():
                e0 = cid * CH
                pltpu.make_async_copy(td_h.at[idxd_v], rowd_v, semd).wait()
                pltpu.make_async_copy(ts_h.at[idxs_v], rows_v, sems).wait()

                def add_row(i, cy):
                    for j in range(sw32 // 16):
                        sl = pl.ds(16 * j, 16)
                        a = plsc.bitcast(rowd_v[i, sl], BF16)
                        b = plsc.bitcast(rows_v[i, sl], BF16)
                        rowd_v[i, sl] = plsc.bitcast(a + b, jnp.int32)
                    return cy

                lax.fori_loop(0, CH, add_row, 0)
                if with_rel:
                    for g_ in range(CH // 16):
                        sv = idxs_v[pl.ds(16 * g_, 16)] * 3
                        dv = idxd_v[pl.ds(16 * g_, 16)] * 3
                        flat = (lanes + 16 * g_) * 4
                        d2 = jnp.zeros((16,), F32)
                        for comp in range(3):
                            xs = plsc.load_gather(x_v, [sv + comp])
                            xd = plsc.load_gather(x_v, [dv + comp])
                            r = xd - xs
                            plsc.store_scatter(rel_v, [flat + comp], r)
                            d2 = d2 + r * r
                        plsc.store_scatter(rel_v, [flat + 3], d2)
                    pltpu.sync_copy(rel_v, r_out.at[pl.ds(e0 * 4, CH * 4)])
                pltpu.sync_copy(rowd_v, g_out.at[pl.ds(e0, CH)])

        issue(b0, wid)

        def pair_body(p, carry):
            k0 = 2 * p
            issue(b1, wid + NW * (k0 + 1))
            process(b0, wid + NW * k0)
            issue(b0, wid + NW * (k0 + 2))
            process(b1, wid + NW * (k0 + 1))
            return carry

        lax.fori_loop(0, npair, pair_body, 0)

    if with_rel:
        @functools.partial(
            pl.kernel,
            mesh=_mesh(),
            compiler_params=pltpu.CompilerParams(needs_layout_passes=False),
            out_type=tuple(outs),
            scratch_types=scratch,
        )
        def body(td_h, ts_h, src_h, dst_h, x_h, g_out, r_out,
                 i0s, i0d, r0d, r0s, s0d, s0s,
                 i1s, i1d, r1d, r1s, s1d, s1s, x_v, rel_v):
            impl(td_h, ts_h, src_h, dst_h, x_h, g_out, r_out,
                 (i0s, i0d, r0d, r0s, s0d, s0s),
                 (i1s, i1d, r1d, r1s, s1d, s1s), x_v, rel_v)

        g32, relflat = body(td, ts, src, dst, xflat)
        return g32, relflat.reshape(N_EDGES, 4)
    else:
        @functools.partial(
            pl.kernel,
            mesh=_mesh(),
            compiler_params=pltpu.CompilerParams(needs_layout_passes=False),
            out_type=tuple(outs),
            scratch_types=scratch,
        )
        def body(td_h, ts_h, src_h, dst_h, g_out,
                 i0s, i0d, r0d, r0s, s0d, s0s,
                 i1s, i1d, r1d, r1s, s1d, s1s):
            impl(td_h, ts_h, src_h, dst_h, None, g_out, None,
                 (i0s, i0d, r0d, r0s, s0d, s0s),
                 (i1s, i1d, r1d, r1s, s1d, s1s), None, None)

        res = body(td, ts, src, dst)
        return res[0] if isinstance(res, (tuple, list)) else res


def _sc_gather_rel(td, ts, src, dst, x):
    return _sc_gather_impl(td, ts, src, dst, x.reshape(-1))


def _sc_gather(td, ts, src, dst):
    return _sc_gather_impl(td, ts, src, dst, None)


def _sc_scatter_add(data, dst, width):
    zeros = jnp.zeros((ACC_ROWS, width), F32)
    rz = ACC_ROWS // NTILE
    rpt = 320
    tail = HALF - (NTILE - 1) * rpt

    @functools.partial(
        pl.kernel,
        mesh=_mesh(),
        compiler_params=pltpu.CompilerParams(needs_layout_passes=False,
                                             use_tc_tiling_on_sc=False),
        out_type=jax.ShapeDtypeStruct((NSC, HALF, width), F32),
        scratch_types=[pltpu.VMEM((CH,), jnp.int32),
                       pltpu.VMEM((CH,), jnp.int32),
                       pltpu.VMEM((CH, width), F32),
                       pltpu.VMEM_SHARED((ACC_ROWS, width), F32),
                       pltpu.SemaphoreType.DMA],
    )
    def body(data_h, dst_h, z_h, out_h, idxd_v, idxl_v, data_v, acc_sh, sem):
        c = lax.axis_index("c")
        s = lax.axis_index("s")
        base = c * HALF
        pltpu.sync_copy(z_h.at[pl.ds(rz * s, rz)], acc_sh.at[pl.ds(rz * s, rz)])
        plsc.subcore_barrier()

        def chunk_body(k, carry):
            cid = s + NTILE * k

            @pl.when(cid < NCHUNK)
            def _():
                e0 = cid * CH
                pltpu.sync_copy(dst_h.at[pl.ds(e0, CH)], idxd_v)
                pltpu.sync_copy(data_h.at[pl.ds(e0, CH)], data_v)
                for g_ in range(CH // 16):
                    sl = pl.ds(16 * g_, 16)
                    loc = idxd_v[sl] - base
                    ok = (loc >= 0) & (loc < HALF)
                    idxl_v[sl] = jnp.where(ok, loc, HALF)
                pltpu.sync_copy(data_v, acc_sh.at[idxl_v], add=True)
            return carry

        lax.fori_loop(0, (NCHUNK + NTILE - 1) // NTILE, chunk_body, 0)
        plsc.subcore_barrier()

        @pl.when(s < NTILE - 1)
        def _():
            pltpu.sync_copy(acc_sh.at[pl.ds(rpt * s, rpt)],
                            out_h.at[c, pl.ds(rpt * s, rpt)])

        @pl.when(s == NTILE - 1)
        def _():
            pltpu.sync_copy(acc_sh.at[pl.ds(rpt * s, tail)],
                            out_h.at[c, pl.ds(rpt * s, tail)])

    return body(data, dst, zeros).reshape(N_NODES, width)


def _r_feat(ea, rel):
    d2 = rel[:, 3:4]
    dist = jnp.sqrt(d2)
    offs = lax.broadcasted_iota(jnp.int32, (1, NRG), 1).astype(F32) * _GS_STEP
    df = jnp.exp(_GS_COEFF * (dist - offs) ** 2)
    rf = jnp.concatenate([ea[:, a:a + 1] * df for a in range(EFD)], axis=1)
    return rf


def _edge1_body(g_ref, ea_ref, rel_ref, wer, ghk, behk, ghv, behv,
                w1hk, b1hk, w1hv, b1hv, eww, ewb, prep, psum, s1_ref):
    lo, hi = _unpack(g_ref[:])
    gp = jnp.concatenate([lo[:, :SW2], hi[:, :SW2]], axis=1).astype(F32)
    gq = jnp.concatenate([lo[:, SW2:], hi[:, SW2:]], axis=1).astype(F32)
    ea = ea_ref[:]
    rel = rel_ref[:]
    rf = _r_feat(ea, rel)
    er = jnp.concatenate([ea, rf], axis=1)
    pre = er @ wer[:] + gp
    k = _ln_relu(pre[:, :HIDDEN], ghk[:], behk[:]) @ w1hk[:] + b1hk[:]
    v = _ln_relu(pre[:, HIDDEN:], ghv[:], behv[:]) @ w1hv[:] + b1hv[:]
    ew = jax.nn.sigmoid(rf @ eww[:] + ewb[:])
    v = v * ew
    logits = ((gq * k) @ psum[:]) * 0.25
    ex = jnp.exp(logits)
    s1_ref[:] = jnp.concatenate([(ex @ prep[:]) * v, ex], axis=1)


def _tc_edge1(g1, ea, rel, wer, ghk, behk, ghv, behv, w1hk, b1hk,
              w1hv, b1hv, eww, ewb, prep, psum):
    return pl.pallas_call(
        _edge1_body,
        grid=(N_EDGES // EB,),
        in_specs=[_row((EB, GW2)), _row((EB, EFD)), _row((EB, 4)),
                  _bc((ER, SW)), _bc((1, HIDDEN)), _bc((1, HIDDEN)),
                  _bc((1, HIDDEN)), _bc((1, HIDDEN)),
                  _bc((HIDDEN, HIDDEN)), _bc((1, HIDDEN)),
                  _bc((HIDDEN, HIDDEN)), _bc((1, HIDDEN)),
                  _bc((RFEAT, 1)), _bc((1, 1)),
                  _bc((HEADS, HIDDEN)), _bc((HIDDEN, HEADS))],
        out_specs=_row((EB, S1W)),
        out_shape=jax.ShapeDtypeStruct((N_EDGES, S1W), F32),
    )(g1, ea, rel, wer, ghk, behk, ghv, behv, w1hk, b1hk, w1hv, b1hv,
      eww, ewb, prep, psum)


def _edge2_body(g_ref, ea_ref, rel_ref, wer, gxk, bexk, gxv, bexv,
                w1xk, b1xk, w1xv, b1xv, eww, ewb, psum, s2_ref):
    lo, hi = _unpack(g_ref[:])
    gp = jnp.concatenate([lo[:, :SW2], hi[:, :SW2]], axis=1).astype(F32)
    gq = jnp.concatenate([lo[:, SW2:], hi[:, SW2:]], axis=1).astype(F32)
    ea = ea_ref[:]
    rel = rel_ref[:]
    rf = _r_feat(ea, rel)
    er = jnp.concatenate([ea, rf], axis=1)
    pre = er @ wer[:] + gp
    k = _ln_relu(pre[:, :HIDDEN], gxk[:], bexk[:]) @ w1xk[:] + b1xk[:]
    v = _ln_relu(pre[:, HIDDEN:], gxv[:], bexv[:]) @ w1xv[:] + b1xv[:]
    ew = jax.nn.sigmoid(rf @ eww[:] + ewb[:])
    v = v * ew
    logits = ((gq * k) @ psum[:]) * 0.25
    ex = jnp.exp(logits)
    w = ex * v
    s2_ref[:] = jnp.concatenate([w * rel[:, 0:1], w * rel[:, 1:2],
                                 w * rel[:, 2:3], ex], axis=1)


def _tc_edge2(g2, ea, rel, wer, gxk, bexk, gxv, bexv, w1xk, b1xk,
              w1xv, b1xv, eww, ewb, psum):
    return pl.pallas_call(
        _edge2_body,
        grid=(N_EDGES // EB,),
        in_specs=[_row((EB, GW2)), _row((EB, EFD)), _row((EB, 4)),
                  _bc((ER, SW)), _bc((1, HIDDEN)), _bc((1, HIDDEN)),
                  _bc((1, HIDDEN)), _bc((1, HIDDEN)),
                  _bc((HIDDEN, HIDDEN)), _bc((1, HIDDEN)),
                  _bc((HIDDEN, HEADS)), _bc((1, HEADS)),
                  _bc((RFEAT, 1)), _bc((1, 1)), _bc((HIDDEN, HEADS))],
        out_specs=_row((EB, S2W)),
        out_shape=jax.ShapeDtypeStruct((N_EDGES, S2W), F32),
    )(g2, ea, rel, wer, gxk, bexk, gxv, bexv, w1xk, b1xk, w1xv, b1xv,
      eww, ewb, psum)


def _node_out_body(acc_ref, h_ref, w0no, b0no, gno, beno, w1no, b1no, prep,
                   wq0, bq0, gq, beq, wq1, bq1, wi, wj, b0c,
                   hout_ref, td_ref, ts_ref):
    a = acc_ref[:]
    h = h_ref[:]
    den = (a[:, HIDDEN:] @ prep[:]) + 1e-16
    attn = a[:, :HIDDEN] / den
    z = jnp.concatenate([attn, h], axis=1) @ w0no[:] + b0no[:]
    hout = _ln_relu(z, gno[:], beno[:]) @ w1no[:] + b1no[:] + h
    hout_ref[:] = hout
    q = _ln_relu(hout @ wq0[:] + bq0[:], gq[:], beq[:]) @ wq1[:] + bq1[:]
    u = hout @ wi[:] + b0c[:]
    s = hout @ wj[:]
    td_ref[:] = _pack_rows(u, q)
    ts_ref[:] = _pack(s[:, :SW2], s[:, SW2:])


def _tc_node_out(acc1, h, w0no, b0no, gno, beno, w1no, b1no, prep,
                 wq0, bq0, gq, beq, wq1, bq1, wi, wj, b0c):
    return pl.pallas_call(
        _node_out_body,
        grid=(N_NODES // NB,),
        in_specs=[_row((NB, S1W)), _row((NB, HIDDEN)),
                  _bc((2 * HIDDEN, HIDDEN)), _bc((1, HIDDEN)),
                  _bc((1, HIDDEN)), _bc((1, HIDDEN)),
                  _bc((HIDDEN, HIDDEN)), _bc((1, HIDDEN)),
                  _bc((HEADS, HIDDEN)),
                  _bc((HIDDEN, HIDDEN)), _bc((1, HIDDEN)), _bc((1, HIDDEN)),
                  _bc((1, HIDDEN)), _bc((HIDDEN, HIDDEN)), _bc((1, HIDDEN)),
                  _bc((HIDDEN, SW)), _bc((HIDDEN, SW)), _bc((1, SW))],
        out_specs=(_row((NB, HIDDEN)), _row((NB, GW2)), _row((NB, SW2))),
        out_shape=(jax.ShapeDtypeStruct((N_NODES, HIDDEN), F32),
                   jax.ShapeDtypeStruct((N_NODES, GW2), jnp.int32),
                   jax.ShapeDtypeStruct((N_NODES, SW2), jnp.int32)),
    )(acc1, h, w0no, b0no, gno, beno, w1no, b1no, prep,
      wq0, bq0, gq, beq, wq1, bq1, wi, wj, b0c)


def _finalize_body(acc_ref, x_ref, m_ref, xo_ref):
    a = acc_ref[:]
    inv = 1.0 / (a[:, 3 * HEADS:] + 1e-16)
    one = jnp.ones((HEADS, 1), F32)
    scale = 1.0 / HEADS
    parts = [((a[:, c * HEADS:(c + 1) * HEADS] * inv) @ one) * scale
             for c in range(3)]
    delta = jnp.concatenate(parts, axis=1)
    xo_ref[:] = x_ref[:] + delta * m_ref[:]


def _tc_finalize(acc2, x, mask):
    return pl.pallas_call(
        _finalize_body,
        grid=(N_NODES // NB,),
        in_specs=[_row((NB, S2W)), _row((NB, 3)), _row((NB, 1))],
        out_specs=_row((NB, 3)),
        out_shape=jax.ShapeDtypeStruct((N_NODES, 3), F32),
    )(acc2, x, mask)


def _split_kv(w0):
    return w0[:ER], w0[ER:ER + HIDDEN], w0[ER + HIDDEN:]


def kernel(h, x, edge_attr, edge_index, mask_ligand, params):
    src = edge_index[0]
    dst = edge_index[1]
    p1 = params["x2h"]
    p2 = params["h2x"]

    er_hk, wi_hk, wj_hk = _split_kv(p1["hk"]["w0"])
    er_hv, wi_hv, wj_hv = _split_kv(p1["hv"]["w0"])
    wi1 = jnp.concatenate([wi_hk, wi_hv], axis=1)
    wj1 = jnp.concatenate([wj_hk, wj_hv], axis=1)
    wer1 = jnp.concatenate([er_hk, er_hv], axis=1)
    b01 = jnp.concatenate([p1["hk"]["b0"], p1["hv"]["b0"]])[None, :]

    er_xk, wi_xk, wj_xk = _split_kv(p2["xk"]["w0"])
    er_xv, wi_xv, wj_xv = _split_kv(p2["xv"]["w0"])
    wi2 = jnp.concatenate([wi_xk, wi_xv], axis=1)
    wj2 = jnp.concatenate([wj_xk, wj_xv], axis=1)
    wer2 = jnp.concatenate([er_xk, er_xv], axis=1)
    b02 = jnp.concatenate([p2["xk"]["b0"], p2["xv"]["b0"]])[None, :]

    prep = jnp.asarray(np.kron(np.eye(HEADS, dtype=np.float32),
                               np.ones((1, HEAD_DIM), np.float32)))
    psum = prep.T

    def r1(v):
        return v[None, :]

    hq = p1["hq"]
    td1, ts1 = _tc_node_pre(h, hq["w0"], r1(hq["b0"]), r1(hq["g"]),
                            r1(hq["be"]), hq["w1"], r1(hq["b1"]),
                            wi1, wj1, b01)

    g1, rel = _sc_gather_rel(td1, ts1, src, dst, x)

    s1 = _tc_edge1(g1, edge_attr, rel, wer1,
                   r1(p1["hk"]["g"]), r1(p1["hk"]["be"]),
                   r1(p1["hv"]["g"]), r1(p1["hv"]["be"]),
                   p1["hk"]["w1"], r1(p1["hk"]["b1"]),
                   p1["hv"]["w1"], r1(p1["hv"]["b1"]),
                   p1["ew_w"], p1["ew_b"][None, :], prep, psum)

    acc1 = _sc_scatter_add(s1, dst, S1W)

    no = p1["node_out"]
    xq = p2["xq"]
    h_out, td2, ts2 = _tc_node_out(
        acc1, h, no["w0"], r1(no["b0"]), r1(no["g"]), r1(no["be"]),
        no["w1"], r1(no["b1"]), prep,
        xq["w0"], r1(xq["b0"]), r1(xq["g"]), r1(xq["be"]),
        xq["w1"], r1(xq["b1"]), wi2, wj2, b02)

    g2 = _sc_gather(td2, ts2, src, dst)

    s2 = _tc_edge2(g2, edge_attr, rel, wer2,
                   r1(p2["xk"]["g"]), r1(p2["xk"]["be"]),
                   r1(p2["xv"]["g"]), r1(p2["xv"]["be"]),
                   p2["xk"]["w1"], r1(p2["xk"]["b1"]),
                   p2["xv"]["w1"], r1(p2["xv"]["b1"]),
                   p2["ew_w"], p2["ew_b"][None, :], psum)

    acc2 = _sc_scatter_add(s2, dst, S2W)

    x_out = _tc_finalize(acc2, x, mask_ligand[:, None])
    return h_out, x_out


# speedup vs baseline: 2.8260x; 1.1655x over previous
import functools

import jax
import jax.numpy as jnp
import numpy as np
from jax import lax
from jax.experimental import pallas as pl
from jax.experimental.pallas import tpu as pltpu
from jax.experimental.pallas import tpu_sc as plsc

F32 = jnp.float32
BF16 = jnp.bfloat16
HIDDEN = 256
HEADS = 16
HEAD_DIM = HIDDEN // HEADS
NRG = 20
EFD = 4
RFEAT = NRG * 4
ER = EFD + RFEAT
N_NODES = 10000
N_EDGES = 160000

NB = 400
EB = 640
CH = 64
EH = N_EDGES // 2
NSC = 2
NTILE = 16
NW = NSC * NTILE
HALF = N_NODES // NSC
ACC_ROWS = HALF + 120
GW = 3 * HIDDEN
SW = 2 * HIDDEN
GW2 = GW // 2
SW2 = SW // 2
S1W = HIDDEN + HEADS
S2W = 4 * HEADS

_GS_STEP = 10.0 / (NRG - 1)
_GS_COEFF = -0.5 / (_GS_STEP * _GS_STEP)


def _mesh():
    return plsc.VectorSubcoreMesh(core_axis_name="c", subcore_axis_name="s",
                                  num_cores=NSC, num_subcores=NTILE)


def _ln_relu(y, g, be):
    mu = jnp.mean(y, axis=-1, keepdims=True)
    yc = y - mu
    var = jnp.mean(yc * yc, axis=-1, keepdims=True)
    return jnp.maximum(yc * lax.rsqrt(var + 1e-5) * g + be, 0.0)


def _pack(a, b):
    au = lax.bitcast_convert_type(a.astype(BF16), jnp.uint16).astype(jnp.uint32)
    bu = lax.bitcast_convert_type(b.astype(BF16), jnp.uint16).astype(jnp.uint32)
    return lax.bitcast_convert_type(au | (bu << 16), jnp.int32)


def _unpack(g):
    gu = lax.bitcast_convert_type(g, jnp.uint32)
    lo = lax.bitcast_convert_type((gu & 0xFFFF).astype(jnp.uint16), BF16)
    hi = lax.bitcast_convert_type((gu >> 16).astype(jnp.uint16), BF16)
    return lo, hi


def _bc(shape):
    return pl.BlockSpec(shape, lambda i: (0,) * len(shape))


def _row(shape):
    return pl.BlockSpec(shape, lambda i: (i,) + (0,) * (len(shape) - 1))


def _pack_rows(u, q):
    return jnp.concatenate(
        [_pack(u[:, :SW2], u[:, SW2:]),
         _pack(q[:, :HIDDEN // 2], q[:, HIDDEN // 2:])], axis=1)


def _node_pre_body(h_ref, wq0, bq0, gq, beq, wq1, bq1, wi, wj, b0c,
                   td_ref, ts_ref):
    h = h_ref[:]
    q = _ln_relu(h @ wq0[:] + bq0[:], gq[:], beq[:]) @ wq1[:] + bq1[:]
    u = h @ wi[:] + b0c[:]
    s = h @ wj[:]
    td_ref[:] = _pack_rows(u, q)
    ts_ref[:] = _pack(s[:, :SW2], s[:, SW2:])


def _tc_node_pre(h, wq0, bq0, gq, beq, wq1, bq1, wi, wj, b0c):
    return pl.pallas_call(
        _node_pre_body,
        grid=(N_NODES // NB,),
        in_specs=[_row((NB, HIDDEN)), _bc((HIDDEN, HIDDEN)), _bc((1, HIDDEN)),
                  _bc((1, HIDDEN)), _bc((1, HIDDEN)), _bc((HIDDEN, HIDDEN)),
                  _bc((1, HIDDEN)), _bc((HIDDEN, SW)), _bc((HIDDEN, SW)),
                  _bc((1, SW))],
        out_specs=(_row((NB, GW2)), _row((NB, SW2))),
        out_shape=(jax.ShapeDtypeStruct((N_NODES, GW2), jnp.int32),
                   jax.ShapeDtypeStruct((N_NODES, SW2), jnp.int32)),
    )(h, wq0, bq0, gq, beq, wq1, bq1, wi, wj, b0c)


def _sc_gather_impl(td, ts, src, dst, xflat):
    with_rel = xflat is not None
    n_edges = src.shape[0]
    nchunks = n_edges // CH
    npair = ((nchunks + NW - 1) // NW + 1) // 2
    gw32 = GW2
    sw32 = SW2

    outs = [jax.ShapeDtypeStruct((n_edges, gw32), jnp.int32)]
    scratch = [pltpu.VMEM((CH,), jnp.int32), pltpu.VMEM((CH,), jnp.int32),
               pltpu.VMEM((CH, gw32), jnp.int32), pltpu.VMEM((CH, sw32), jnp.int32),
               pltpu.SemaphoreType.DMA, pltpu.SemaphoreType.DMA,
               pltpu.VMEM((CH,), jnp.int32), pltpu.VMEM((CH,), jnp.int32),
               pltpu.VMEM((CH, gw32), jnp.int32), pltpu.VMEM((CH, sw32), jnp.int32),
               pltpu.SemaphoreType.DMA, pltpu.SemaphoreType.DMA]
    if with_rel:
        outs.append(jax.ShapeDtypeStruct((n_edges * 4,), F32))
        scratch += [pltpu.VMEM((N_NODES * 3,), F32),
                    pltpu.VMEM((CH * 4,), F32)]

    def impl(td_h, ts_h, src_h, dst_h, x_h, g_out, r_out,
             b0, b1, x_v, rel_v):
        c = lax.axis_index("c")
        s = lax.axis_index("s")
        wid = s * NSC + c
        if with_rel:
            pltpu.sync_copy(x_h, x_v)
        lanes = lax.iota(jnp.int32, 16)

        def issue(buf, cid):
            idxs_v, idxd_v, rowd_v, rows_v, semd, sems = buf

            @pl.when(cid < nchunks)
            def _():
                e0 = cid * CH
                pltpu.sync_copy(src_h.at[pl.ds(e0, CH)], idxs_v)
                pltpu.sync_copy(dst_h.at[pl.ds(e0, CH)], idxd_v)
                pltpu.async_copy(td_h.at[idxd_v], rowd_v, semd)
                pltpu.async_copy(ts_h.at[idxs_v], rows_v, sems)

        def process(buf, cid):
            idxs_v, idxd_v, rowd_v, rows_v, semd, sems = buf

            @pl.when(cid < nchunks)
            def _():
                e0 = cid * CH
                pltpu.make_async_copy(td_h.at[idxd_v], rowd_v, semd).wait()
                pltpu.make_async_copy(ts_h.at[idxs_v], rows_v, sems).wait()

                def add_row(i, cy):
                    for j in range(sw32 // 16):
                        sl = pl.ds(16 * j, 16)
                        a = plsc.bitcast(rowd_v[i, sl], BF16)
                        b = plsc.bitcast(rows_v[i, sl], BF16)
                        rowd_v[i, sl] = plsc.bitcast(a + b, jnp.int32)
                    return cy

                lax.fori_loop(0, CH, add_row, 0)
                if with_rel:
                    for g_ in range(CH // 16):
                        sv = idxs_v[pl.ds(16 * g_, 16)] * 3
                        dv = idxd_v[pl.ds(16 * g_, 16)] * 3
                        flat = (lanes + 16 * g_) * 4
                        d2 = jnp.zeros((16,), F32)
                        for comp in range(3):
                            xs = plsc.load_gather(x_v, [sv + comp])
                            xd = plsc.load_gather(x_v, [dv + comp])
                            r = xd - xs
                            plsc.store_scatter(rel_v, [flat + comp], r)
                            d2 = d2 + r * r
                        plsc.store_scatter(rel_v, [flat + 3], d2)
                    pltpu.sync_copy(rel_v, r_out.at[pl.ds(e0 * 4, CH * 4)])
                pltpu.sync_copy(rowd_v, g_out.at[pl.ds(e0, CH)])

        issue(b0, wid)

        def pair_body(p, carry):
            k0 = 2 * p
            issue(b1, wid + NW * (k0 + 1))
            process(b0, wid + NW * k0)
            issue(b0, wid + NW * (k0 + 2))
            process(b1, wid + NW * (k0 + 1))
            return carry

        lax.fori_loop(0, npair, pair_body, 0)

    if with_rel:
        @functools.partial(
            pl.kernel,
            mesh=_mesh(),
            compiler_params=pltpu.CompilerParams(needs_layout_passes=False),
            out_type=tuple(outs),
            scratch_types=scratch,
        )
        def body(td_h, ts_h, src_h, dst_h, x_h, g_out, r_out,
                 i0s, i0d, r0d, r0s, s0d, s0s,
                 i1s, i1d, r1d, r1s, s1d, s1s, x_v, rel_v):
            impl(td_h, ts_h, src_h, dst_h, x_h, g_out, r_out,
                 (i0s, i0d, r0d, r0s, s0d, s0s),
                 (i1s, i1d, r1d, r1s, s1d, s1s), x_v, rel_v)

        g32, relflat = body(td, ts, src, dst, xflat)
        return g32, relflat.reshape(n_edges, 4)
    else:
        @functools.partial(
            pl.kernel,
            mesh=_mesh(),
            compiler_params=pltpu.CompilerParams(needs_layout_passes=False),
            out_type=tuple(outs),
            scratch_types=scratch,
        )
        def body(td_h, ts_h, src_h, dst_h, g_out,
                 i0s, i0d, r0d, r0s, s0d, s0s,
                 i1s, i1d, r1d, r1s, s1d, s1s):
            impl(td_h, ts_h, src_h, dst_h, None, g_out, None,
                 (i0s, i0d, r0d, r0s, s0d, s0s),
                 (i1s, i1d, r1d, r1s, s1d, s1s), None, None)

        res = body(td, ts, src, dst)
        return res[0] if isinstance(res, (tuple, list)) else res


def _sc_gather_rel(td, ts, src, dst, x):
    return _sc_gather_impl(td, ts, src, dst, x.reshape(-1))


def _sc_gather(td, ts, src, dst):
    return _sc_gather_impl(td, ts, src, dst, None)


def _sc_scatter_add(data_a, data_b, dst, width):
    zeros = jnp.zeros((ACC_ROWS, width), F32)
    rz = ACC_ROWS // NTILE
    rpt = 320
    tail = HALF - (NTILE - 1) * rpt
    nch = EH // CH

    @functools.partial(
        pl.kernel,
        mesh=_mesh(),
        compiler_params=pltpu.CompilerParams(needs_layout_passes=False,
                                             use_tc_tiling_on_sc=False),
        out_type=jax.ShapeDtypeStruct((NSC, HALF, width), F32),
        scratch_types=[pltpu.VMEM((CH,), jnp.int32),
                       pltpu.VMEM((CH,), jnp.int32),
                       pltpu.VMEM((CH, width), F32),
                       pltpu.VMEM_SHARED((ACC_ROWS, width), F32),
                       pltpu.SemaphoreType.DMA],
    )
    def body(da_h, db_h, dst_h, z_h, out_h, idxd_v, idxl_v, data_v,
             acc_sh, sem):
        c = lax.axis_index("c")
        s = lax.axis_index("s")
        base = c * HALF
        pltpu.sync_copy(z_h.at[pl.ds(rz * s, rz)], acc_sh.at[pl.ds(rz * s, rz)])
        plsc.subcore_barrier()

        def half_loop(data_h, e_off):
            def chunk_body(k, carry):
                cid = s + NTILE * k

                @pl.when(cid < nch)
                def _():
                    e0 = cid * CH
                    pltpu.sync_copy(dst_h.at[pl.ds(e_off + e0, CH)], idxd_v)
                    pltpu.sync_copy(data_h.at[pl.ds(e0, CH)], data_v)
                    for g_ in range(CH // 16):
                        sl = pl.ds(16 * g_, 16)
                        loc = idxd_v[sl] - base
                        ok = (loc >= 0) & (loc < HALF)
                        idxl_v[sl] = jnp.where(ok, loc, HALF)
                    pltpu.sync_copy(data_v, acc_sh.at[idxl_v], add=True)
                return carry

            lax.fori_loop(0, (nch + NTILE - 1) // NTILE, chunk_body, 0)

        half_loop(da_h, 0)
        half_loop(db_h, EH)
        plsc.subcore_barrier()

        @pl.when(s < NTILE - 1)
        def _():
            pltpu.sync_copy(acc_sh.at[pl.ds(rpt * s, rpt)],
                            out_h.at[c, pl.ds(rpt * s, rpt)])

        @pl.when(s == NTILE - 1)
        def _():
            pltpu.sync_copy(acc_sh.at[pl.ds(rpt * s, tail)],
                            out_h.at[c, pl.ds(rpt * s, tail)])

    return body(data_a, data_b, dst, zeros).reshape(N_NODES, width)


def _r_feat(ea, rel):
    d2 = rel[:, 3:4]
    dist = jnp.sqrt(d2)
    offs = lax.broadcasted_iota(jnp.int32, (1, NRG), 1).astype(F32) * _GS_STEP
    df = jnp.exp(_GS_COEFF * (dist - offs) ** 2)
    rf = jnp.concatenate([ea[:, a:a + 1] * df for a in range(EFD)], axis=1)
    return rf


def _edge1_body(g_ref, ea_ref, rel_ref, wer, ghk, behk, ghv, behv,
                w1hk, b1hk, w1hv, b1hv, eww, ewb, prep, psum, s1_ref):
    lo, hi = _unpack(g_ref[:])
    gp = jnp.concatenate([lo[:, :SW2], hi[:, :SW2]], axis=1).astype(F32)
    gq = jnp.concatenate([lo[:, SW2:], hi[:, SW2:]], axis=1).astype(F32)
    ea = ea_ref[:]
    rel = rel_ref[:]
    rf = _r_feat(ea, rel)
    er = jnp.concatenate([ea, rf], axis=1)
    pre = er @ wer[:] + gp
    k = _ln_relu(pre[:, :HIDDEN], ghk[:], behk[:]) @ w1hk[:] + b1hk[:]
    v = _ln_relu(pre[:, HIDDEN:], ghv[:], behv[:]) @ w1hv[:] + b1hv[:]
    ew = jax.nn.sigmoid(rf @ eww[:] + ewb[:])
    v = v * ew
    logits = ((gq * k) @ psum[:]) * 0.25
    ex = jnp.exp(logits)
    s1_ref[:] = jnp.concatenate([(ex @ prep[:]) * v, ex], axis=1)


def _tc_edge1(g1, ea, rel, wer, ghk, behk, ghv, behv, w1hk, b1hk,
              w1hv, b1hv, eww, ewb, prep, psum):
    return pl.pallas_call(
        _edge1_body,
        grid=(g1.shape[0] // EB,),
        in_specs=[_row((EB, GW2)), _row((EB, EFD)), _row((EB, 4)),
                  _bc((ER, SW)), _bc((1, HIDDEN)), _bc((1, HIDDEN)),
                  _bc((1, HIDDEN)), _bc((1, HIDDEN)),
                  _bc((HIDDEN, HIDDEN)), _bc((1, HIDDEN)),
                  _bc((HIDDEN, HIDDEN)), _bc((1, HIDDEN)),
                  _bc((RFEAT, 1)), _bc((1, 1)),
                  _bc((HEADS, HIDDEN)), _bc((HIDDEN, HEADS))],
        out_specs=_row((EB, S1W)),
        out_shape=jax.ShapeDtypeStruct((g1.shape[0], S1W), F32),
    )(g1, ea, rel, wer, ghk, behk, ghv, behv, w1hk, b1hk, w1hv, b1hv,
      eww, ewb, prep, psum)


def _edge2_body(g_ref, ea_ref, rel_ref, wer, gxk, bexk, gxv, bexv,
                w1xk, b1xk, w1xv, b1xv, eww, ewb, psum, s2_ref):
    lo, hi = _unpack(g_ref[:])
    gp = jnp.concatenate([lo[:, :SW2], hi[:, :SW2]], axis=1).astype(F32)
    gq = jnp.concatenate([lo[:, SW2:], hi[:, SW2:]], axis=1).astype(F32)
    ea = ea_ref[:]
    rel = rel_ref[:]
    rf = _r_feat(ea, rel)
    er = jnp.concatenate([ea, rf], axis=1)
    pre = er @ wer[:] + gp
    k = _ln_relu(pre[:, :HIDDEN], gxk[:], bexk[:]) @ w1xk[:] + b1xk[:]
    v = _ln_relu(pre[:, HIDDEN:], gxv[:], bexv[:]) @ w1xv[:] + b1xv[:]
    ew = jax.nn.sigmoid(rf @ eww[:] + ewb[:])
    v = v * ew
    logits = ((gq * k) @ psum[:]) * 0.25
    ex = jnp.exp(logits)
    w = ex * v
    s2_ref[:] = jnp.concatenate([w * rel[:, 0:1], w * rel[:, 1:2],
                                 w * rel[:, 2:3], ex], axis=1)


def _tc_edge2(g2, ea, rel, wer, gxk, bexk, gxv, bexv, w1xk, b1xk,
              w1xv, b1xv, eww, ewb, psum):
    return pl.pallas_call(
        _edge2_body,
        grid=(g2.shape[0] // EB,),
        in_specs=[_row((EB, GW2)), _row((EB, EFD)), _row((EB, 4)),
                  _bc((ER, SW)), _bc((1, HIDDEN)), _bc((1, HIDDEN)),
                  _bc((1, HIDDEN)), _bc((1, HIDDEN)),
                  _bc((HIDDEN, HIDDEN)), _bc((1, HIDDEN)),
                  _bc((HIDDEN, HEADS)), _bc((1, HEADS)),
                  _bc((RFEAT, 1)), _bc((1, 1)), _bc((HIDDEN, HEADS))],
        out_specs=_row((EB, S2W)),
        out_shape=jax.ShapeDtypeStruct((g2.shape[0], S2W), F32),
    )(g2, ea, rel, wer, gxk, bexk, gxv, bexv, w1xk, b1xk, w1xv, b1xv,
      eww, ewb, psum)


def _node_out_body(acc_ref, h_ref, w0no, b0no, gno, beno, w1no, b1no, prep,
                   wq0, bq0, gq, beq, wq1, bq1, wi, wj, b0c,
                   hout_ref, td_ref, ts_ref):
    a = acc_ref[:]
    h = h_ref[:]
    den = (a[:, HIDDEN:] @ prep[:]) + 1e-16
    attn = a[:, :HIDDEN] / den
    z = jnp.concatenate([attn, h], axis=1) @ w0no[:] + b0no[:]
    hout = _ln_relu(z, gno[:], beno[:]) @ w1no[:] + b1no[:] + h
    hout_ref[:] = hout
    q = _ln_relu(hout @ wq0[:] + bq0[:], gq[:], beq[:]) @ wq1[:] + bq1[:]
    u = hout @ wi[:] + b0c[:]
    s = hout @ wj[:]
    td_ref[:] = _pack_rows(u, q)
    ts_ref[:] = _pack(s[:, :SW2], s[:, SW2:])


def _tc_node_out(acc1, h, w0no, b0no, gno, beno, w1no, b1no, prep,
                 wq0, bq0, gq, beq, wq1, bq1, wi, wj, b0c):
    return pl.pallas_call(
        _node_out_body,
        grid=(N_NODES // NB,),
        in_specs=[_row((NB, S1W)), _row((NB, HIDDEN)),
                  _bc((2 * HIDDEN, HIDDEN)), _bc((1, HIDDEN)),
                  _bc((1, HIDDEN)), _bc((1, HIDDEN)),
                  _bc((HIDDEN, HIDDEN)), _bc((1, HIDDEN)),
                  _bc((HEADS, HIDDEN)),
                  _bc((HIDDEN, HIDDEN)), _bc((1, HIDDEN)), _bc((1, HIDDEN)),
                  _bc((1, HIDDEN)), _bc((HIDDEN, HIDDEN)), _bc((1, HIDDEN)),
                  _bc((HIDDEN, SW)), _bc((HIDDEN, SW)), _bc((1, SW))],
        out_specs=(_row((NB, HIDDEN)), _row((NB, GW2)), _row((NB, SW2))),
        out_shape=(jax.ShapeDtypeStruct((N_NODES, HIDDEN), F32),
                   jax.ShapeDtypeStruct((N_NODES, GW2), jnp.int32),
                   jax.ShapeDtypeStruct((N_NODES, SW2), jnp.int32)),
    )(acc1, h, w0no, b0no, gno, beno, w1no, b1no, prep,
      wq0, bq0, gq, beq, wq1, bq1, wi, wj, b0c)


def _finalize_body(acc_ref, x_ref, m_ref, xo_ref):
    a = acc_ref[:]
    inv = 1.0 / (a[:, 3 * HEADS:] + 1e-16)
    one = jnp.ones((HEADS, 1), F32)
    scale = 1.0 / HEADS
    parts = [((a[:, c * HEADS:(c + 1) * HEADS] * inv) @ one) * scale
             for c in range(3)]
    delta = jnp.concatenate(parts, axis=1)
    xo_ref[:] = x_ref[:] + delta * m_ref[:]


def _tc_finalize(acc2, x, mask):
    return pl.pallas_call(
        _finalize_body,
        grid=(N_NODES // NB,),
        in_specs=[_row((NB, S2W)), _row((NB, 3)), _row((NB, 1))],
        out_specs=_row((NB, 3)),
        out_shape=jax.ShapeDtypeStruct((N_NODES, 3), F32),
    )(acc2, x, mask)


def _split_kv(w0):
    return w0[:ER], w0[ER:ER + HIDDEN], w0[ER + HIDDEN:]


def kernel(h, x, edge_attr, edge_index, mask_ligand, params):
    src = edge_index[0]
    dst = edge_index[1]
    p1 = params["x2h"]
    p2 = params["h2x"]

    er_hk, wi_hk, wj_hk = _split_kv(p1["hk"]["w0"])
    er_hv, wi_hv, wj_hv = _split_kv(p1["hv"]["w0"])
    wi1 = jnp.concatenate([wi_hk, wi_hv], axis=1)
    wj1 = jnp.concatenate([wj_hk, wj_hv], axis=1)
    wer1 = jnp.concatenate([er_hk, er_hv], axis=1)
    b01 = jnp.concatenate([p1["hk"]["b0"], p1["hv"]["b0"]])[None, :]

    er_xk, wi_xk, wj_xk = _split_kv(p2["xk"]["w0"])
    er_xv, wi_xv, wj_xv = _split_kv(p2["xv"]["w0"])
    wi2 = jnp.concatenate([wi_xk, wi_xv], axis=1)
    wj2 = jnp.concatenate([wj_xk, wj_xv], axis=1)
    wer2 = jnp.concatenate([er_xk, er_xv], axis=1)
    b02 = jnp.concatenate([p2["xk"]["b0"], p2["xv"]["b0"]])[None, :]

    prep = jnp.asarray(np.kron(np.eye(HEADS, dtype=np.float32),
                               np.ones((1, HEAD_DIM), np.float32)))
    psum = prep.T

    def r1(v):
        return v[None, :]

    hq = p1["hq"]
    td1, ts1 = _tc_node_pre(h, hq["w0"], r1(hq["b0"]), r1(hq["g"]),
                            r1(hq["be"]), hq["w1"], r1(hq["b1"]),
                            wi1, wj1, b01)

    src_a, src_b = src[:EH], src[EH:]
    dst_a, dst_b = dst[:EH], dst[EH:]
    ea_a, ea_b = edge_attr[:EH], edge_attr[EH:]

    def edge1(g, ea, rel):
        return _tc_edge1(g, ea, rel, wer1,
                         r1(p1["hk"]["g"]), r1(p1["hk"]["be"]),
                         r1(p1["hv"]["g"]), r1(p1["hv"]["be"]),
                         p1["hk"]["w1"], r1(p1["hk"]["b1"]),
                         p1["hv"]["w1"], r1(p1["hv"]["b1"]),
                         p1["ew_w"], p1["ew_b"][None, :], prep, psum)

    g1a, rel_a = _sc_gather_rel(td1, ts1, src_a, dst_a, x)
    g1b, rel_b = _sc_gather_rel(td1, ts1, src_b, dst_b, x)
    s1a = edge1(g1a, ea_a, rel_a)
    s1b = edge1(g1b, ea_b, rel_b)

    acc1 = _sc_scatter_add(s1a, s1b, dst, S1W)

    no = p1["node_out"]
    xq = p2["xq"]
    h_out, td2, ts2 = _tc_node_out(
        acc1, h, no["w0"], r1(no["b0"]), r1(no["g"]), r1(no["be"]),
        no["w1"], r1(no["b1"]), prep,
        xq["w0"], r1(xq["b0"]), r1(xq["g"]), r1(xq["be"]),
        xq["w1"], r1(xq["b1"]), wi2, wj2, b02)

    def edge2(g, ea, rel):
        return _tc_edge2(g, ea, rel, wer2,
                         r1(p2["xk"]["g"]), r1(p2["xk"]["be"]),
                         r1(p2["xv"]["g"]), r1(p2["xv"]["be"]),
                         p2["xk"]["w1"], r1(p2["xk"]["b1"]),
                         p2["xv"]["w1"], r1(p2["xv"]["b1"]),
                         p2["ew_w"], p2["ew_b"][None, :], psum)

    g2a = _sc_gather(td2, ts2, src_a, dst_a)
    g2b = _sc_gather(td2, ts2, src_b, dst_b)
    s2a = edge2(g2a, ea_a, rel_a)
    s2b = edge2(g2b, ea_b, rel_b)

    acc2 = _sc_scatter_add(s2a, s2b, dst, S2W)

    x_out = _tc_finalize(acc2, x, mask_ligand[:, None])
    return h_out, x_out


# speedup vs baseline: 3.1223x; 1.1048x over previous
import functools

import jax
import jax.numpy as jnp
import numpy as np
from jax import lax
from jax.experimental import pallas as pl
from jax.experimental.pallas import tpu as pltpu
from jax.experimental.pallas import tpu_sc as plsc

F32 = jnp.float32
BF16 = jnp.bfloat16
HIDDEN = 256
HEADS = 16
HEAD_DIM = HIDDEN // HEADS
NRG = 20
EFD = 4
RFEAT = NRG * 4
ER = EFD + RFEAT
N_NODES = 10000
N_EDGES = 160000

NB = 400
EB = 640
CH = 64
EH = N_EDGES // 2
NSC = 2
NTILE = 16
NW = NSC * NTILE
HALF = N_NODES // NSC
ACC_ROWS = HALF + 120
GW = 3 * HIDDEN
SW = 2 * HIDDEN
GW2 = GW // 2
SW2 = SW // 2
S1W = HIDDEN + HEADS
S2W = 4 * HEADS

_GS_STEP = 10.0 / (NRG - 1)
_GS_COEFF = -0.5 / (_GS_STEP * _GS_STEP)


def _mesh():
    return plsc.VectorSubcoreMesh(core_axis_name="c", subcore_axis_name="s",
                                  num_cores=NSC, num_subcores=NTILE)


def _ln_relu(y, g, be):
    mu = jnp.mean(y, axis=-1, keepdims=True)
    yc = y - mu
    var = jnp.mean(yc * yc, axis=-1, keepdims=True)
    return jnp.maximum(yc * lax.rsqrt(var + 1e-5) * g + be, 0.0)


def _pack(a, b):
    au = lax.bitcast_convert_type(a.astype(BF16), jnp.uint16).astype(jnp.uint32)
    bu = lax.bitcast_convert_type(b.astype(BF16), jnp.uint16).astype(jnp.uint32)
    return lax.bitcast_convert_type(au | (bu << 16), jnp.int32)


def _unpack(g):
    gu = lax.bitcast_convert_type(g, jnp.uint32)
    lo = lax.bitcast_convert_type((gu & 0xFFFF).astype(jnp.uint16), BF16)
    hi = lax.bitcast_convert_type((gu >> 16).astype(jnp.uint16), BF16)
    return lo, hi


def _bc(shape):
    return pl.BlockSpec(shape, lambda i: (0,) * len(shape))


def _row(shape):
    return pl.BlockSpec(shape, lambda i: (i,) + (0,) * (len(shape) - 1))


def _pack_rows(u, q):
    return jnp.concatenate(
        [_pack(u[:, :SW2], u[:, SW2:]),
         _pack(q[:, :HIDDEN // 2], q[:, HIDDEN // 2:])], axis=1)


def _node_pre_body(h_ref, wq0, bq0, gq, beq, wq1, bq1, wi, wj, b0c,
                   td_ref, ts_ref):
    h = h_ref[:]
    q = _ln_relu(h @ wq0[:] + bq0[:], gq[:], beq[:]) @ wq1[:] + bq1[:]
    u = h @ wi[:] + b0c[:]
    s = h @ wj[:]
    td_ref[:] = _pack_rows(u, q)
    ts_ref[:] = _pack(s[:, :SW2], s[:, SW2:])


def _tc_node_pre(h, wq0, bq0, gq, beq, wq1, bq1, wi, wj, b0c):
    return pl.pallas_call(
        _node_pre_body,
        grid=(N_NODES // NB,),
        in_specs=[_row((NB, HIDDEN)), _bc((HIDDEN, HIDDEN)), _bc((1, HIDDEN)),
                  _bc((1, HIDDEN)), _bc((1, HIDDEN)), _bc((HIDDEN, HIDDEN)),
                  _bc((1, HIDDEN)), _bc((HIDDEN, SW)), _bc((HIDDEN, SW)),
                  _bc((1, SW))],
        out_specs=(_row((NB, GW2)), _row((NB, SW2))),
        out_shape=(jax.ShapeDtypeStruct((N_NODES, GW2), jnp.int32),
                   jax.ShapeDtypeStruct((N_NODES, SW2), jnp.int32)),
    )(h, wq0, bq0, gq, beq, wq1, bq1, wi, wj, b0c)


def _sc_gather_impl(td, ts, src, dst, xflat):
    with_rel = xflat is not None
    n_edges = src.shape[0]
    nchunks = n_edges // CH
    npair = ((nchunks + NW - 1) // NW + 1) // 2
    gw32 = GW2
    sw32 = SW2

    outs = [jax.ShapeDtypeStruct((n_edges, gw32), jnp.int32)]
    scratch = [pltpu.VMEM((CH,), jnp.int32), pltpu.VMEM((CH,), jnp.int32),
               pltpu.VMEM((CH, gw32), jnp.int32), pltpu.VMEM((CH, sw32), jnp.int32),
               pltpu.SemaphoreType.DMA, pltpu.SemaphoreType.DMA,
               pltpu.VMEM((CH,), jnp.int32), pltpu.VMEM((CH,), jnp.int32),
               pltpu.VMEM((CH, gw32), jnp.int32), pltpu.VMEM((CH, sw32), jnp.int32),
               pltpu.SemaphoreType.DMA, pltpu.SemaphoreType.DMA]
    if with_rel:
        outs.append(jax.ShapeDtypeStruct((n_edges * 4,), F32))
        scratch += [pltpu.VMEM((N_NODES * 3,), F32),
                    pltpu.VMEM((CH * 4,), F32)]

    def impl(td_h, ts_h, src_h, dst_h, x_h, g_out, r_out,
             b0, b1, x_v, rel_v):
        c = lax.axis_index("c")
        s = lax.axis_index("s")
        wid = s * NSC + c
        if with_rel:
            pltpu.sync_copy(x_h, x_v)
        lanes = lax.iota(jnp.int32, 16)

        def issue(buf, cid):
            idxs_v, idxd_v, rowd_v, rows_v, semd, sems = buf

            @pl.when(cid < nchunks)
            def _():
                e0 = cid * CH
                pltpu.sync_copy(src_h.at[pl.ds(e0, CH)], idxs_v)
                pltpu.sync_copy(dst_h.at[pl.ds(e0, CH)], idxd_v)
                pltpu.async_copy(td_h.at[idxd_v], rowd_v, semd)
                pltpu.async_copy(ts_h.at[idxs_v], rows_v, sems)

        def process(buf, cid):
            idxs_v, idxd_v, rowd_v, rows_v, semd, sems = buf

            @pl.when(cid < nchunks)
            def _():
                e0 = cid * CH
                pltpu.make_async_copy(td_h.at[idxd_v], rowd_v, semd).wait()
                pltpu.make_async_copy(ts_h.at[idxs_v], rows_v, sems).wait()

                def add_row(i, cy):
                    for j in range(sw32 // 16):
                        sl = pl.ds(16 * j, 16)
                        a = plsc.bitcast(rowd_v[i, sl], BF16)
                        b = plsc.bitcast(rows_v[i, sl], BF16)
                        rowd_v[i, sl] = plsc.bitcast(a + b, jnp.int32)
                    return cy

                lax.fori_loop(0, CH, add_row, 0)
                if with_rel:
                    for g_ in range(CH // 16):
                        sv = idxs_v[pl.ds(16 * g_, 16)] * 3
                        dv = idxd_v[pl.ds(16 * g_, 16)] * 3
                        flat = (lanes + 16 * g_) * 4
                        d2 = jnp.zeros((16,), F32)
                        for comp in range(3):
                            xs = plsc.load_gather(x_v, [sv + comp])
                            xd = plsc.load_gather(x_v, [dv + comp])
                            r = xd - xs
                            plsc.store_scatter(rel_v, [flat + comp], r)
                            d2 = d2 + r * r
                        plsc.store_scatter(rel_v, [flat + 3], d2)
                    pltpu.sync_copy(rel_v, r_out.at[pl.ds(e0 * 4, CH * 4)])
                pltpu.sync_copy(rowd_v, g_out.at[pl.ds(e0, CH)])

        issue(b0, wid)

        def pair_body(p, carry):
            k0 = 2 * p
            issue(b1, wid + NW * (k0 + 1))
            process(b0, wid + NW * k0)
            issue(b0, wid + NW * (k0 + 2))
            process(b1, wid + NW * (k0 + 1))
            return carry

        lax.fori_loop(0, npair, pair_body, 0)

    if with_rel:
        @functools.partial(
            pl.kernel,
            mesh=_mesh(),
            compiler_params=pltpu.CompilerParams(needs_layout_passes=False),
            out_type=tuple(outs),
            scratch_types=scratch,
        )
        def body(td_h, ts_h, src_h, dst_h, x_h, g_out, r_out,
                 i0s, i0d, r0d, r0s, s0d, s0s,
                 i1s, i1d, r1d, r1s, s1d, s1s, x_v, rel_v):
            impl(td_h, ts_h, src_h, dst_h, x_h, g_out, r_out,
                 (i0s, i0d, r0d, r0s, s0d, s0s),
                 (i1s, i1d, r1d, r1s, s1d, s1s), x_v, rel_v)

        g32, relflat = body(td, ts, src, dst, xflat)
        return g32, relflat.reshape(n_edges, 4)
    else:
        @functools.partial(
            pl.kernel,
            mesh=_mesh(),
            compiler_params=pltpu.CompilerParams(needs_layout_passes=False),
            out_type=tuple(outs),
            scratch_types=scratch,
        )
        def body(td_h, ts_h, src_h, dst_h, g_out,
                 i0s, i0d, r0d, r0s, s0d, s0s,
                 i1s, i1d, r1d, r1s, s1d, s1s):
            impl(td_h, ts_h, src_h, dst_h, None, g_out, None,
                 (i0s, i0d, r0d, r0s, s0d, s0s),
                 (i1s, i1d, r1d, r1s, s1d, s1s), None, None)

        res = body(td, ts, src, dst)
        return res[0] if isinstance(res, (tuple, list)) else res


def _sc_gather_rel(td, ts, src, dst, x):
    return _sc_gather_impl(td, ts, src, dst, x.reshape(-1))


def _sc_gather(td, ts, src, dst):
    return _sc_gather_impl(td, ts, src, dst, None)


def _sc_scatter_add(data, dst_half, init, width):
    rz = ACC_ROWS // NTILE
    nch = data.shape[0] // CH

    @functools.partial(
        pl.kernel,
        mesh=_mesh(),
        compiler_params=pltpu.CompilerParams(needs_layout_passes=False,
                                             use_tc_tiling_on_sc=False),
        out_type=jax.ShapeDtypeStruct((NSC, ACC_ROWS, width), F32),
        scratch_types=[pltpu.VMEM((CH,), jnp.int32),
                       pltpu.VMEM((CH,), jnp.int32),
                       pltpu.VMEM((CH, width), F32),
                       pltpu.VMEM_SHARED((ACC_ROWS, width), F32),
                       pltpu.SemaphoreType.DMA],
    )
    def body(d_h, dst_h, init_h, out_h, idxd_v, idxl_v, data_v,
             acc_sh, sem):
        c = lax.axis_index("c")
        s = lax.axis_index("s")
        base = c * HALF
        pltpu.sync_copy(init_h.at[c, pl.ds(rz * s, rz)],
                        acc_sh.at[pl.ds(rz * s, rz)])
        plsc.subcore_barrier()

        def chunk_body(k, carry):
            cid = s + NTILE * k

            @pl.when(cid < nch)
            def _():
                e0 = cid * CH
                pltpu.sync_copy(dst_h.at[pl.ds(e0, CH)], idxd_v)
                pltpu.sync_copy(d_h.at[pl.ds(e0, CH)], data_v)
                for g_ in range(CH // 16):
                    sl = pl.ds(16 * g_, 16)
                    loc = idxd_v[sl] - base
                    ok = (loc >= 0) & (loc < HALF)
                    idxl_v[sl] = jnp.where(ok, loc, HALF)
                pltpu.sync_copy(data_v, acc_sh.at[idxl_v], add=True)
            return carry

        lax.fori_loop(0, (nch + NTILE - 1) // NTILE, chunk_body, 0)
        plsc.subcore_barrier()

        pltpu.sync_copy(acc_sh.at[pl.ds(rz * s, rz)],
                        out_h.at[c, pl.ds(rz * s, rz)])

    return body(data, dst_half, init)


def _sc_scatter_add2(data_a, data_b, dst_a, dst_b, width):
    zeros = jnp.zeros((NSC, ACC_ROWS, width), F32)
    acc_a = _sc_scatter_add(data_a, dst_a, zeros, width)
    acc_b = _sc_scatter_add(data_b, dst_b, acc_a, width)
    return acc_b[:, :HALF].reshape(N_NODES, width)


def _r_feat(ea, rel):
    d2 = rel[:, 3:4]
    dist = jnp.sqrt(d2)
    offs = lax.broadcasted_iota(jnp.int32, (1, NRG), 1).astype(F32) * _GS_STEP
    df = jnp.exp(_GS_COEFF * (dist - offs) ** 2)
    rf = jnp.concatenate([ea[:, a:a + 1] * df for a in range(EFD)], axis=1)
    return rf


def _edge1_body(g_ref, ea_ref, rel_ref, wer, ghk, behk, ghv, behv,
                w1hk, b1hk, w1hv, b1hv, eww, ewb, prep, psum, s1_ref):
    lo, hi = _unpack(g_ref[:])
    gp = jnp.concatenate([lo[:, :SW2], hi[:, :SW2]], axis=1).astype(F32)
    gq = jnp.concatenate([lo[:, SW2:], hi[:, SW2:]], axis=1).astype(F32)
    ea = ea_ref[:]
    rel = rel_ref[:]
    rf = _r_feat(ea, rel)
    er = jnp.concatenate([ea, rf], axis=1)
    pre = er @ wer[:] + gp
    k = _ln_relu(pre[:, :HIDDEN], ghk[:], behk[:]) @ w1hk[:] + b1hk[:]
    v = _ln_relu(pre[:, HIDDEN:], ghv[:], behv[:]) @ w1hv[:] + b1hv[:]
    ew = jax.nn.sigmoid(rf @ eww[:] + ewb[:])
    v = v * ew
    logits = ((gq * k) @ psum[:]) * 0.25
    ex = jnp.exp(logits)
    s1_ref[:] = jnp.concatenate([(ex @ prep[:]) * v, ex], axis=1)


def _tc_edge1(g1, ea, rel, wer, ghk, behk, ghv, behv, w1hk, b1hk,
              w1hv, b1hv, eww, ewb, prep, psum):
    return pl.pallas_call(
        _edge1_body,
        grid=(g1.shape[0] // EB,),
        in_specs=[_row((EB, GW2)), _row((EB, EFD)), _row((EB, 4)),
                  _bc((ER, SW)), _bc((1, HIDDEN)), _bc((1, HIDDEN)),
                  _bc((1, HIDDEN)), _bc((1, HIDDEN)),
                  _bc((HIDDEN, HIDDEN)), _bc((1, HIDDEN)),
                  _bc((HIDDEN, HIDDEN)), _bc((1, HIDDEN)),
                  _bc((RFEAT, 1)), _bc((1, 1)),
                  _bc((HEADS, HIDDEN)), _bc((HIDDEN, HEADS))],
        out_specs=_row((EB, S1W)),
        out_shape=jax.ShapeDtypeStruct((g1.shape[0], S1W), F32),
    )(g1, ea, rel, wer, ghk, behk, ghv, behv, w1hk, b1hk, w1hv, b1hv,
      eww, ewb, prep, psum)


def _edge2_body(g_ref, ea_ref, rel_ref, wer, gxk, bexk, gxv, bexv,
                w1xk, b1xk, w1xv, b1xv, eww, ewb, psum, s2_ref):
    lo, hi = _unpack(g_ref[:])
    gp = jnp.concatenate([lo[:, :SW2], hi[:, :SW2]], axis=1).astype(F32)
    gq = jnp.concatenate([lo[:, SW2:], hi[:, SW2:]], axis=1).astype(F32)
    ea = ea_ref[:]
    rel = rel_ref[:]
    rf = _r_feat(ea, rel)
    er = jnp.concatenate([ea, rf], axis=1)
    pre = er @ wer[:] + gp
    k = _ln_relu(pre[:, :HIDDEN], gxk[:], bexk[:]) @ w1xk[:] + b1xk[:]
    v = _ln_relu(pre[:, HIDDEN:], gxv[:], bexv[:]) @ w1xv[:] + b1xv[:]
    ew = jax.nn.sigmoid(rf @ eww[:] + ewb[:])
    v = v * ew
    logits = ((gq * k) @ psum[:]) * 0.25
    ex = jnp.exp(logits)
    w = ex * v
    s2_ref[:] = jnp.concatenate([w * rel[:, 0:1], w * rel[:, 1:2],
                                 w * rel[:, 2:3], ex], axis=1)


def _tc_edge2(g2, ea, rel, wer, gxk, bexk, gxv, bexv, w1xk, b1xk,
              w1xv, b1xv, eww, ewb, psum):
    return pl.pallas_call(
        _edge2_body,
        grid=(g2.shape[0] // EB,),
        in_specs=[_row((EB, GW2)), _row((EB, EFD)), _row((EB, 4)),
                  _bc((ER, SW)), _bc((1, HIDDEN)), _bc((1, HIDDEN)),
                  _bc((1, HIDDEN)), _bc((1, HIDDEN)),
                  _bc((HIDDEN, HIDDEN)), _bc((1, HIDDEN)),
                  _bc((HIDDEN, HEADS)), _bc((1, HEADS)),
                  _bc((RFEAT, 1)), _bc((1, 1)), _bc((HIDDEN, HEADS))],
        out_specs=_row((EB, S2W)),
        out_shape=jax.ShapeDtypeStruct((g2.shape[0], S2W), F32),
    )(g2, ea, rel, wer, gxk, bexk, gxv, bexv, w1xk, b1xk, w1xv, b1xv,
      eww, ewb, psum)


def _node_out_body(acc_ref, h_ref, w0no, b0no, gno, beno, w1no, b1no, prep,
                   wq0, bq0, gq, beq, wq1, bq1, wi, wj, b0c,
                   hout_ref, td_ref, ts_ref):
    a = acc_ref[:]
    h = h_ref[:]
    den = (a[:, HIDDEN:] @ prep[:]) + 1e-16
    attn = a[:, :HIDDEN] / den
    z = jnp.concatenate([attn, h], axis=1) @ w0no[:] + b0no[:]
    hout = _ln_relu(z, gno[:], beno[:]) @ w1no[:] + b1no[:] + h
    hout_ref[:] = hout
    q = _ln_relu(hout @ wq0[:] + bq0[:], gq[:], beq[:]) @ wq1[:] + bq1[:]
    u = hout @ wi[:] + b0c[:]
    s = hout @ wj[:]
    td_ref[:] = _pack_rows(u, q)
    ts_ref[:] = _pack(s[:, :SW2], s[:, SW2:])


def _tc_node_out(acc1, h, w0no, b0no, gno, beno, w1no, b1no, prep,
                 wq0, bq0, gq, beq, wq1, bq1, wi, wj, b0c):
    return pl.pallas_call(
        _node_out_body,
        grid=(N_NODES // NB,),
        in_specs=[_row((NB, S1W)), _row((NB, HIDDEN)),
                  _bc((2 * HIDDEN, HIDDEN)), _bc((1, HIDDEN)),
                  _bc((1, HIDDEN)), _bc((1, HIDDEN)),
                  _bc((HIDDEN, HIDDEN)), _bc((1, HIDDEN)),
                  _bc((HEADS, HIDDEN)),
                  _bc((HIDDEN, HIDDEN)), _bc((1, HIDDEN)), _bc((1, HIDDEN)),
                  _bc((1, HIDDEN)), _bc((HIDDEN, HIDDEN)), _bc((1, HIDDEN)),
                  _bc((HIDDEN, SW)), _bc((HIDDEN, SW)), _bc((1, SW))],
        out_specs=(_row((NB, HIDDEN)), _row((NB, GW2)), _row((NB, SW2))),
        out_shape=(jax.ShapeDtypeStruct((N_NODES, HIDDEN), F32),
                   jax.ShapeDtypeStruct((N_NODES, GW2), jnp.int32),
                   jax.ShapeDtypeStruct((N_NODES, SW2), jnp.int32)),
    )(acc1, h, w0no, b0no, gno, beno, w1no, b1no, prep,
      wq0, bq0, gq, beq, wq1, bq1, wi, wj, b0c)


def _finalize_body(acc_ref, x_ref, m_ref, xo_ref):
    a = acc_ref[:]
    inv = 1.0 / (a[:, 3 * HEADS:] + 1e-16)
    one = jnp.ones((HEADS, 1), F32)
    scale = 1.0 / HEADS
    parts = [((a[:, c * HEADS:(c + 1) * HEADS] * inv) @ one) * scale
             for c in range(3)]
    delta = jnp.concatenate(parts, axis=1)
    xo_ref[:] = x_ref[:] + delta * m_ref[:]


def _tc_finalize(acc2, x, mask):
    return pl.pallas_call(
        _finalize_body,
        grid=(N_NODES // NB,),
        in_specs=[_row((NB, S2W)), _row((NB, 3)), _row((NB, 1))],
        out_specs=_row((NB, 3)),
        out_shape=jax.ShapeDtypeStruct((N_NODES, 3), F32),
    )(acc2, x, mask)


def _split_kv(w0):
    return w0[:ER], w0[ER:ER + HIDDEN], w0[ER + HIDDEN:]


def kernel(h, x, edge_attr, edge_index, mask_ligand, params):
    src = edge_index[0]
    dst = edge_index[1]
    p1 = params["x2h"]
    p2 = params["h2x"]

    er_hk, wi_hk, wj_hk = _split_kv(p1["hk"]["w0"])
    er_hv, wi_hv, wj_hv = _split_kv(p1["hv"]["w0"])
    wi1 = jnp.concatenate([wi_hk, wi_hv], axis=1)
    wj1 = jnp.concatenate([wj_hk, wj_hv], axis=1)
    wer1 = jnp.concatenate([er_hk, er_hv], axis=1)
    b01 = jnp.concatenate([p1["hk"]["b0"], p1["hv"]["b0"]])[None, :]

    er_xk, wi_xk, wj_xk = _split_kv(p2["xk"]["w0"])
    er_xv, wi_xv, wj_xv = _split_kv(p2["xv"]["w0"])
    wi2 = jnp.concatenate([wi_xk, wi_xv], axis=1)
    wj2 = jnp.concatenate([wj_xk, wj_xv], axis=1)
    wer2 = jnp.concatenate([er_xk, er_xv], axis=1)
    b02 = jnp.concatenate([p2["xk"]["b0"], p2["xv"]["b0"]])[None, :]

    prep = jnp.asarray(np.kron(np.eye(HEADS, dtype=np.float32),
                               np.ones((1, HEAD_DIM), np.float32)))
    psum = prep.T

    def r1(v):
        return v[None, :]

    hq = p1["hq"]
    td1, ts1 = _tc_node_pre(h, hq["w0"], r1(hq["b0"]), r1(hq["g"]),
                            r1(hq["be"]), hq["w1"], r1(hq["b1"]),
                            wi1, wj1, b01)

    src_a, src_b = src[:EH], src[EH:]
    dst_a, dst_b = dst[:EH], dst[EH:]
    ea_a, ea_b = edge_attr[:EH], edge_attr[EH:]

    def edge1(g, ea, rel):
        return _tc_edge1(g, ea, rel, wer1,
                         r1(p1["hk"]["g"]), r1(p1["hk"]["be"]),
                         r1(p1["hv"]["g"]), r1(p1["hv"]["be"]),
                         p1["hk"]["w1"], r1(p1["hk"]["b1"]),
                         p1["hv"]["w1"], r1(p1["hv"]["b1"]),
                         p1["ew_w"], p1["ew_b"][None, :], prep, psum)

    g1a, rel_a = _sc_gather_rel(td1, ts1, src_a, dst_a, x)
    g1b, rel_b = _sc_gather_rel(td1, ts1, src_b, dst_b, x)
    s1a = edge1(g1a, ea_a, rel_a)
    s1b = edge1(g1b, ea_b, rel_b)

    acc1 = _sc_scatter_add2(s1a, s1b, dst_a, dst_b, S1W)

    no = p1["node_out"]
    xq = p2["xq"]
    h_out, td2, ts2 = _tc_node_out(
        acc1, h, no["w0"], r1(no["b0"]), r1(no["g"]), r1(no["be"]),
        no["w1"], r1(no["b1"]), prep,
        xq["w0"], r1(xq["b0"]), r1(xq["g"]), r1(xq["be"]),
        xq["w1"], r1(xq["b1"]), wi2, wj2, b02)

    def edge2(g, ea, rel):
        return _tc_edge2(g, ea, rel, wer2,
                         r1(p2["xk"]["g"]), r1(p2["xk"]["be"]),
                         r1(p2["xv"]["g"]), r1(p2["xv"]["be"]),
                         p2["xk"]["w1"], r1(p2["xk"]["b1"]),
                         p2["xv"]["w1"], r1(p2["xv"]["b1"]),
                         p2["ew_w"], p2["ew_b"][None, :], psum)

    g2a = _sc_gather(td2, ts2, src_a, dst_a)
    g2b = _sc_gather(td2, ts2, src_b, dst_b)
    s2a = edge2(g2a, ea_a, rel_a)
    s2b = edge2(g2b, ea_b, rel_b)

    acc2 = _sc_scatter_add2(s2a, s2b, dst_a, dst_b, S2W)

    x_out = _tc_finalize(acc2, x, mask_ligand[:, None])
    return h_out, x_out


# speedup vs baseline: 3.3564x; 1.0750x over previous
import functools

import jax
import jax.numpy as jnp
import numpy as np
from jax import lax
from jax.experimental import pallas as pl
from jax.experimental.pallas import tpu as pltpu
from jax.experimental.pallas import tpu_sc as plsc

F32 = jnp.float32
BF16 = jnp.bfloat16
HIDDEN = 256
HEADS = 16
HEAD_DIM = HIDDEN // HEADS
NRG = 20
EFD = 4
RFEAT = NRG * 4
ER = EFD + RFEAT
N_NODES = 10000
N_EDGES = 160000

NB = 400
EB = 640
CH = 64
NPART = 5
NSC = 2
NTILE = 16
NW = NSC * NTILE
HALF = N_NODES // NSC
ACC_ROWS = HALF + 120
GW = 3 * HIDDEN
SW = 2 * HIDDEN
GW2 = GW // 2
SW2 = SW // 2
S1W = HIDDEN + HEADS
S2W = 4 * HEADS

_GS_STEP = 10.0 / (NRG - 1)
_GS_COEFF = -0.5 / (_GS_STEP * _GS_STEP)


def _mesh():
    return plsc.VectorSubcoreMesh(core_axis_name="c", subcore_axis_name="s",
                                  num_cores=NSC, num_subcores=NTILE)


def _ln_relu(y, g, be):
    mu = jnp.mean(y, axis=-1, keepdims=True)
    yc = y - mu
    var = jnp.mean(yc * yc, axis=-1, keepdims=True)
    return jnp.maximum(yc * lax.rsqrt(var + 1e-5) * g + be, 0.0)


def _pack(a, b):
    au = lax.bitcast_convert_type(a.astype(BF16), jnp.uint16).astype(jnp.uint32)
    bu = lax.bitcast_convert_type(b.astype(BF16), jnp.uint16).astype(jnp.uint32)
    return lax.bitcast_convert_type(au | (bu << 16), jnp.int32)


def _unpack(g):
    gu = lax.bitcast_convert_type(g, jnp.uint32)
    lo = lax.bitcast_convert_type((gu & 0xFFFF).astype(jnp.uint16), BF16)
    hi = lax.bitcast_convert_type((gu >> 16).astype(jnp.uint16), BF16)
    return lo, hi


def _bc(shape):
    return pl.BlockSpec(shape, lambda i: (0,) * len(shape))


def _row(shape):
    return pl.BlockSpec(shape, lambda i: (i,) + (0,) * (len(shape) - 1))


def _pack_rows(u, q):
    return jnp.concatenate(
        [_pack(u[:, :SW2], u[:, SW2:]),
         _pack(q[:, :HIDDEN // 2], q[:, HIDDEN // 2:])], axis=1)


def _node_pre_body(h_ref, wq0, bq0, gq, beq, wq1, bq1, wi, wj, b0c,
                   td_ref, ts_ref):
    h = h_ref[:]
    q = _ln_relu(h @ wq0[:] + bq0[:], gq[:], beq[:]) @ wq1[:] + bq1[:]
    u = h @ wi[:] + b0c[:]
    s = h @ wj[:]
    td_ref[:] = _pack_rows(u, q)
    ts_ref[:] = _pack(s[:, :SW2], s[:, SW2:])


def _tc_node_pre(h, wq0, bq0, gq, beq, wq1, bq1, wi, wj, b0c):
    return pl.pallas_call(
        _node_pre_body,
        grid=(N_NODES // NB,),
        in_specs=[_row((NB, HIDDEN)), _bc((HIDDEN, HIDDEN)), _bc((1, HIDDEN)),
                  _bc((1, HIDDEN)), _bc((1, HIDDEN)), _bc((HIDDEN, HIDDEN)),
                  _bc((1, HIDDEN)), _bc((HIDDEN, SW)), _bc((HIDDEN, SW)),
                  _bc((1, SW))],
        out_specs=(_row((NB, GW2)), _row((NB, SW2))),
        out_shape=(jax.ShapeDtypeStruct((N_NODES, GW2), jnp.int32),
                   jax.ShapeDtypeStruct((N_NODES, SW2), jnp.int32)),
    )(h, wq0, bq0, gq, beq, wq1, bq1, wi, wj, b0c)


def _sc_gather_impl(td, ts, src, dst, xflat):
    with_rel = xflat is not None
    n_edges = src.shape[0]
    nchunks = n_edges // CH
    npair = ((nchunks + NW - 1) // NW + 1) // 2
    gw32 = GW2
    sw32 = SW2

    outs = [jax.ShapeDtypeStruct((n_edges, gw32), jnp.int32)]
    scratch = [pltpu.VMEM((CH,), jnp.int32), pltpu.VMEM((CH,), jnp.int32),
               pltpu.VMEM((CH, gw32), jnp.int32), pltpu.VMEM((CH, sw32), jnp.int32),
               pltpu.SemaphoreType.DMA, pltpu.SemaphoreType.DMA,
               pltpu.VMEM((CH,), jnp.int32), pltpu.VMEM((CH,), jnp.int32),
               pltpu.VMEM((CH, gw32), jnp.int32), pltpu.VMEM((CH, sw32), jnp.int32),
               pltpu.SemaphoreType.DMA, pltpu.SemaphoreType.DMA]
    if with_rel:
        outs.append(jax.ShapeDtypeStruct((n_edges * 4,), F32))
        scratch += [pltpu.VMEM((N_NODES * 3,), F32),
                    pltpu.VMEM((CH * 4,), F32)]

    def impl(td_h, ts_h, src_h, dst_h, x_h, g_out, r_out,
             b0, b1, x_v, rel_v):
        c = lax.axis_index("c")
        s = lax.axis_index("s")
        wid = s * NSC + c
        if with_rel:
            pltpu.sync_copy(x_h, x_v)
        lanes = lax.iota(jnp.int32, 16)

        def issue(buf, cid):
            idxs_v, idxd_v, rowd_v, rows_v, semd, sems = buf

            @pl.when(cid < nchunks)
            def _():
                e0 = cid * CH
                pltpu.sync_copy(src_h.at[pl.ds(e0, CH)], idxs_v)
                pltpu.sync_copy(dst_h.at[pl.ds(e0, CH)], idxd_v)
                pltpu.async_copy(td_h.at[idxd_v], rowd_v, semd)
                pltpu.async_copy(ts_h.at[idxs_v], rows_v, sems)

        def process(buf, cid):
            idxs_v, idxd_v, rowd_v, rows_v, semd, sems = buf

            @pl.when(cid < nchunks)
            def _():
                e0 = cid * CH
                pltpu.make_async_copy(td_h.at[idxd_v], rowd_v, semd).wait()
                pltpu.make_async_copy(ts_h.at[idxs_v], rows_v, sems).wait()

                def add_row(i, cy):
                    for j in range(sw32 // 16):
                        sl = pl.ds(16 * j, 16)
                        a = plsc.bitcast(rowd_v[i, sl], BF16)
                        b = plsc.bitcast(rows_v[i, sl], BF16)
                        rowd_v[i, sl] = plsc.bitcast(a + b, jnp.int32)
                    return cy

                lax.fori_loop(0, CH, add_row, 0)
                if with_rel:
                    for g_ in range(CH // 16):
                        sv = idxs_v[pl.ds(16 * g_, 16)] * 3
                        dv = idxd_v[pl.ds(16 * g_, 16)] * 3
                        flat = (lanes + 16 * g_) * 4
                        d2 = jnp.zeros((16,), F32)
                        for comp in range(3):
                            xs = plsc.load_gather(x_v, [sv + comp])
                            xd = plsc.load_gather(x_v, [dv + comp])
                            r = xd - xs
                            plsc.store_scatter(rel_v, [flat + comp], r)
                            d2 = d2 + r * r
                        plsc.store_scatter(rel_v, [flat + 3], d2)
                    pltpu.sync_copy(rel_v, r_out.at[pl.ds(e0 * 4, CH * 4)])
                pltpu.sync_copy(rowd_v, g_out.at[pl.ds(e0, CH)])

        issue(b0, wid)

        def pair_body(p, carry):
            k0 = 2 * p
            issue(b1, wid + NW * (k0 + 1))
            process(b0, wid + NW * k0)
            issue(b0, wid + NW * (k0 + 2))
            process(b1, wid + NW * (k0 + 1))
            return carry

        lax.fori_loop(0, npair, pair_body, 0)

    if with_rel:
        @functools.partial(
            pl.kernel,
            mesh=_mesh(),
            compiler_params=pltpu.CompilerParams(needs_layout_passes=False),
            out_type=tuple(outs),
            scratch_types=scratch,
        )
        def body(td_h, ts_h, src_h, dst_h, x_h, g_out, r_out,
                 i0s, i0d, r0d, r0s, s0d, s0s,
                 i1s, i1d, r1d, r1s, s1d, s1s, x_v, rel_v):
            impl(td_h, ts_h, src_h, dst_h, x_h, g_out, r_out,
                 (i0s, i0d, r0d, r0s, s0d, s0s),
                 (i1s, i1d, r1d, r1s, s1d, s1s), x_v, rel_v)

        g32, relflat = body(td, ts, src, dst, xflat)
        return g32, relflat.reshape(n_edges, 4)
    else:
        @functools.partial(
            pl.kernel,
            mesh=_mesh(),
            compiler_params=pltpu.CompilerParams(needs_layout_passes=False),
            out_type=tuple(outs),
            scratch_types=scratch,
        )
        def body(td_h, ts_h, src_h, dst_h, g_out,
                 i0s, i0d, r0d, r0s, s0d, s0s,
                 i1s, i1d, r1d, r1s, s1d, s1s):
            impl(td_h, ts_h, src_h, dst_h, None, g_out, None,
                 (i0s, i0d, r0d, r0s, s0d, s0s),
                 (i1s, i1d, r1d, r1s, s1d, s1s), None, None)

        res = body(td, ts, src, dst)
        return res[0] if isinstance(res, (tuple, list)) else res


def _sc_gather_rel(td, ts, src, dst, x):
    return _sc_gather_impl(td, ts, src, dst, x.reshape(-1))


def _sc_gather(td, ts, src, dst):
    return _sc_gather_impl(td, ts, src, dst, None)


def _sc_scatter_add(data, dst_half, init, width):
    rz = ACC_ROWS // NTILE
    nch = data.shape[0] // CH

    @functools.partial(
        pl.kernel,
        mesh=_mesh(),
        compiler_params=pltpu.CompilerParams(needs_layout_passes=False,
                                             use_tc_tiling_on_sc=False),
        out_type=jax.ShapeDtypeStruct((NSC, ACC_ROWS, width), F32),
        scratch_types=[pltpu.VMEM((CH,), jnp.int32),
                       pltpu.VMEM((CH,), jnp.int32),
                       pltpu.VMEM((CH, width), F32),
                       pltpu.VMEM_SHARED((ACC_ROWS, width), F32),
                       pltpu.SemaphoreType.DMA],
    )
    def body(d_h, dst_h, init_h, out_h, idxd_v, idxl_v, data_v,
             acc_sh, sem):
        c = lax.axis_index("c")
        s = lax.axis_index("s")
        base = c * HALF
        pltpu.sync_copy(init_h.at[c, pl.ds(rz * s, rz)],
                        acc_sh.at[pl.ds(rz * s, rz)])
        plsc.subcore_barrier()

        def chunk_body(k, carry):
            cid = s + NTILE * k

            @pl.when(cid < nch)
            def _():
                e0 = cid * CH
                pltpu.sync_copy(dst_h.at[pl.ds(e0, CH)], idxd_v)
                pltpu.sync_copy(d_h.at[pl.ds(e0, CH)], data_v)
                for g_ in range(CH // 16):
                    sl = pl.ds(16 * g_, 16)
                    loc = idxd_v[sl] - base
                    ok = (loc >= 0) & (loc < HALF)
                    idxl_v[sl] = jnp.where(ok, loc, HALF)
                pltpu.sync_copy(data_v, acc_sh.at[idxl_v], add=True)
            return carry

        lax.fori_loop(0, (nch + NTILE - 1) // NTILE, chunk_body, 0)
        plsc.subcore_barrier()

        pltpu.sync_copy(acc_sh.at[pl.ds(rz * s, rz)],
                        out_h.at[c, pl.ds(rz * s, rz)])

    return body(data, dst_half, init)


def _sc_scatter_chain(datas, dsts, width):
    acc = jnp.zeros((NSC, ACC_ROWS, width), F32)
    for d, t in zip(datas, dsts):
        acc = _sc_scatter_add(d, t, acc, width)
    return acc[:, :HALF].reshape(N_NODES, width)


def _r_feat(ea, rel):
    d2 = rel[:, 3:4]
    dist = jnp.sqrt(d2)
    offs = lax.broadcasted_iota(jnp.int32, (1, NRG), 1).astype(F32) * _GS_STEP
    df = jnp.exp(_GS_COEFF * (dist - offs) ** 2)
    rf = jnp.concatenate([ea[:, a:a + 1] * df for a in range(EFD)], axis=1)
    return rf


def _edge1_body(g_ref, ea_ref, rel_ref, wer, ghk, behk, ghv, behv,
                w1hk, b1hk, w1hv, b1hv, eww, ewb, prep, psum, s1_ref):
    lo, hi = _unpack(g_ref[:])
    gp = jnp.concatenate([lo[:, :SW2], hi[:, :SW2]], axis=1).astype(F32)
    gq = jnp.concatenate([lo[:, SW2:], hi[:, SW2:]], axis=1).astype(F32)
    ea = ea_ref[:]
    rel = rel_ref[:]
    rf = _r_feat(ea, rel)
    er = jnp.concatenate([ea, rf], axis=1)
    pre = er @ wer[:] + gp
    k = _ln_relu(pre[:, :HIDDEN], ghk[:], behk[:]) @ w1hk[:] + b1hk[:]
    v = _ln_relu(pre[:, HIDDEN:], ghv[:], behv[:]) @ w1hv[:] + b1hv[:]
    ew = jax.nn.sigmoid(rf @ eww[:] + ewb[:])
    v = v * ew
    logits = ((gq * k) @ psum[:]) * 0.25
    ex = jnp.exp(logits)
    s1_ref[:] = jnp.concatenate([(ex @ prep[:]) * v, ex], axis=1)


def _tc_edge1(g1, ea, rel, wer, ghk, behk, ghv, behv, w1hk, b1hk,
              w1hv, b1hv, eww, ewb, prep, psum):
    return pl.pallas_call(
        _edge1_body,
        grid=(g1.shape[0] // EB,),
        in_specs=[_row((EB, GW2)), _row((EB, EFD)), _row((EB, 4)),
                  _bc((ER, SW)), _bc((1, HIDDEN)), _bc((1, HIDDEN)),
                  _bc((1, HIDDEN)), _bc((1, HIDDEN)),
                  _bc((HIDDEN, HIDDEN)), _bc((1, HIDDEN)),
                  _bc((HIDDEN, HIDDEN)), _bc((1, HIDDEN)),
                  _bc((RFEAT, 1)), _bc((1, 1)),
                  _bc((HEADS, HIDDEN)), _bc((HIDDEN, HEADS))],
        out_specs=_row((EB, S1W)),
        out_shape=jax.ShapeDtypeStruct((g1.shape[0], S1W), F32),
    )(g1, ea, rel, wer, ghk, behk, ghv, behv, w1hk, b1hk, w1hv, b1hv,
      eww, ewb, prep, psum)


def _edge2_body(g_ref, ea_ref, rel_ref, wer, gxk, bexk, gxv, bexv,
                w1xk, b1xk, w1xv, b1xv, eww, ewb, psum, s2_ref):
    lo, hi = _unpack(g_ref[:])
    gp = jnp.concatenate([lo[:, :SW2], hi[:, :SW2]], axis=1).astype(F32)
    gq = jnp.concatenate([lo[:, SW2:], hi[:, SW2:]], axis=1).astype(F32)
    ea = ea_ref[:]
    rel = rel_ref[:]
    rf = _r_feat(ea, rel)
    er = jnp.concatenate([ea, rf], axis=1)
    pre = er @ wer[:] + gp
    k = _ln_relu(pre[:, :HIDDEN], gxk[:], bexk[:]) @ w1xk[:] + b1xk[:]
    v = _ln_relu(pre[:, HIDDEN:], gxv[:], bexv[:]) @ w1xv[:] + b1xv[:]
    ew = jax.nn.sigmoid(rf @ eww[:] + ewb[:])
    v = v * ew
    logits = ((gq * k) @ psum[:]) * 0.25
    ex = jnp.exp(logits)
    w = ex * v
    s2_ref[:] = jnp.concatenate([w * rel[:, 0:1], w * rel[:, 1:2],
                                 w * rel[:, 2:3], ex], axis=1)


def _tc_edge2(g2, ea, rel, wer, gxk, bexk, gxv, bexv, w1xk, b1xk,
              w1xv, b1xv, eww, ewb, psum):
    return pl.pallas_call(
        _edge2_body,
        grid=(g2.shape[0] // EB,),
        in_specs=[_row((EB, GW2)), _row((EB, EFD)), _row((EB, 4)),
                  _bc((ER, SW)), _bc((1, HIDDEN)), _bc((1, HIDDEN)),
                  _bc((1, HIDDEN)), _bc((1, HIDDEN)),
                  _bc((HIDDEN, HIDDEN)), _bc((1, HIDDEN)),
                  _bc((HIDDEN, HEADS)), _bc((1, HEADS)),
                  _bc((RFEAT, 1)), _bc((1, 1)), _bc((HIDDEN, HEADS))],
        out_specs=_row((EB, S2W)),
        out_shape=jax.ShapeDtypeStruct((g2.shape[0], S2W), F32),
    )(g2, ea, rel, wer, gxk, bexk, gxv, bexv, w1xk, b1xk, w1xv, b1xv,
      eww, ewb, psum)


def _node_out_body(acc_ref, h_ref, w0no, b0no, gno, beno, w1no, b1no, prep,
                   wq0, bq0, gq, beq, wq1, bq1, wi, wj, b0c,
                   hout_ref, td_ref, ts_ref):
    a = acc_ref[:]
    h = h_ref[:]
    den = (a[:, HIDDEN:] @ prep[:]) + 1e-16
    attn = a[:, :HIDDEN] / den
    z = jnp.concatenate([attn, h], axis=1) @ w0no[:] + b0no[:]
    hout = _ln_relu(z, gno[:], beno[:]) @ w1no[:] + b1no[:] + h
    hout_ref[:] = hout
    q = _ln_relu(hout @ wq0[:] + bq0[:], gq[:], beq[:]) @ wq1[:] + bq1[:]
    u = hout @ wi[:] + b0c[:]
    s = hout @ wj[:]
    td_ref[:] = _pack_rows(u, q)
    ts_ref[:] = _pack(s[:, :SW2], s[:, SW2:])


def _tc_node_out(acc1, h, w0no, b0no, gno, beno, w1no, b1no, prep,
                 wq0, bq0, gq, beq, wq1, bq1, wi, wj, b0c):
    return pl.pallas_call(
        _node_out_body,
        grid=(N_NODES // NB,),
        in_specs=[_row((NB, S1W)), _row((NB, HIDDEN)),
                  _bc((2 * HIDDEN, HIDDEN)), _bc((1, HIDDEN)),
                  _bc((1, HIDDEN)), _bc((1, HIDDEN)),
                  _bc((HIDDEN, HIDDEN)), _bc((1, HIDDEN)),
                  _bc((HEADS, HIDDEN)),
                  _bc((HIDDEN, HIDDEN)), _bc((1, HIDDEN)), _bc((1, HIDDEN)),
                  _bc((1, HIDDEN)), _bc((HIDDEN, HIDDEN)), _bc((1, HIDDEN)),
                  _bc((HIDDEN, SW)), _bc((HIDDEN, SW)), _bc((1, SW))],
        out_specs=(_row((NB, HIDDEN)), _row((NB, GW2)), _row((NB, SW2))),
        out_shape=(jax.ShapeDtypeStruct((N_NODES, HIDDEN), F32),
                   jax.ShapeDtypeStruct((N_NODES, GW2), jnp.int32),
                   jax.ShapeDtypeStruct((N_NODES, SW2), jnp.int32)),
    )(acc1, h, w0no, b0no, gno, beno, w1no, b1no, prep,
      wq0, bq0, gq, beq, wq1, bq1, wi, wj, b0c)


def _finalize_body(acc_ref, x_ref, m_ref, xo_ref):
    a = acc_ref[:]
    inv = 1.0 / (a[:, 3 * HEADS:] + 1e-16)
    one = jnp.ones((HEADS, 1), F32)
    scale = 1.0 / HEADS
    parts = [((a[:, c * HEADS:(c + 1) * HEADS] * inv) @ one) * scale
             for c in range(3)]
    delta = jnp.concatenate(parts, axis=1)
    xo_ref[:] = x_ref[:] + delta * m_ref[:]


def _tc_finalize(acc2, x, mask):
    return pl.pallas_call(
        _finalize_body,
        grid=(N_NODES // NB,),
        in_specs=[_row((NB, S2W)), _row((NB, 3)), _row((NB, 1))],
        out_specs=_row((NB, 3)),
        out_shape=jax.ShapeDtypeStruct((N_NODES, 3), F32),
    )(acc2, x, mask)


def _split_kv(w0):
    return w0[:ER], w0[ER:ER + HIDDEN], w0[ER + HIDDEN:]


def kernel(h, x, edge_attr, edge_index, mask_ligand, params):
    src = edge_index[0]
    dst = edge_index[1]
    p1 = params["x2h"]
    p2 = params["h2x"]

    er_hk, wi_hk, wj_hk = _split_kv(p1["hk"]["w0"])
    er_hv, wi_hv, wj_hv = _split_kv(p1["hv"]["w0"])
    wi1 = jnp.concatenate([wi_hk, wi_hv], axis=1)
    wj1 = jnp.concatenate([wj_hk, wj_hv], axis=1)
    wer1 = jnp.concatenate([er_hk, er_hv], axis=1)
    b01 = jnp.concatenate([p1["hk"]["b0"], p1["hv"]["b0"]])[None, :]

    er_xk, wi_xk, wj_xk = _split_kv(p2["xk"]["w0"])
    er_xv, wi_xv, wj_xv = _split_kv(p2["xv"]["w0"])
    wi2 = jnp.concatenate([wi_xk, wi_xv], axis=1)
    wj2 = jnp.concatenate([wj_xk, wj_xv], axis=1)
    wer2 = jnp.concatenate([er_xk, er_xv], axis=1)
    b02 = jnp.concatenate([p2["xk"]["b0"], p2["xv"]["b0"]])[None, :]

    prep = jnp.asarray(np.kron(np.eye(HEADS, dtype=np.float32),
                               np.ones((1, HEAD_DIM), np.float32)))
    psum = prep.T

    def r1(v):
        return v[None, :]

    hq = p1["hq"]
    td1, ts1 = _tc_node_pre(h, hq["w0"], r1(hq["b0"]), r1(hq["g"]),
                            r1(hq["be"]), hq["w1"], r1(hq["b1"]),
                            wi1, wj1, b01)

    ep = N_EDGES // NPART
    assert ep * NPART == N_EDGES and ep % EB == 0 and ep % CH == 0
    srcs = [src[i * ep:(i + 1) * ep] for i in range(NPART)]
    dsts = [dst[i * ep:(i + 1) * ep] for i in range(NPART)]
    eas = [edge_attr[i * ep:(i + 1) * ep] for i in range(NPART)]

    def edge1(g, ea, rel):
        return _tc_edge1(g, ea, rel, wer1,
                         r1(p1["hk"]["g"]), r1(p1["hk"]["be"]),
                         r1(p1["hv"]["g"]), r1(p1["hv"]["be"]),
                         p1["hk"]["w1"], r1(p1["hk"]["b1"]),
                         p1["hv"]["w1"], r1(p1["hv"]["b1"]),
                         p1["ew_w"], p1["ew_b"][None, :], prep, psum)

    g1s = [_sc_gather_rel(td1, ts1, srcs[i], dsts[i], x)
           for i in range(NPART)]
    rels = [g[1] for g in g1s]
    s1s = [edge1(g1s[i][0], eas[i], rels[i]) for i in range(NPART)]

    acc1 = _sc_scatter_chain(s1s, dsts, S1W)

    no = p1["node_out"]
    xq = p2["xq"]
    h_out, td2, ts2 = _tc_node_out(
        acc1, h, no["w0"], r1(no["b0"]), r1(no["g"]), r1(no["be"]),
        no["w1"], r1(no["b1"]), prep,
        xq["w0"], r1(xq["b0"]), r1(xq["g"]), r1(xq["be"]),
        xq["w1"], r1(xq["b1"]), wi2, wj2, b02)

    def edge2(g, ea, rel):
        return _tc_edge2(g, ea, rel, wer2,
                         r1(p2["xk"]["g"]), r1(p2["xk"]["be"]),
                         r1(p2["xv"]["g"]), r1(p2["xv"]["be"]),
                         p2["xk"]["w1"], r1(p2["xk"]["b1"]),
                         p2["xv"]["w1"], r1(p2["xv"]["b1"]),
                         p2["ew_w"], p2["ew_b"][None, :], psum)

    g2s = [_sc_gather(td2, ts2, srcs[i], dsts[i]) for i in range(NPART)]
    s2s = [edge2(g2s[i], eas[i], rels[i]) for i in range(NPART)]

    acc2 = _sc_scatter_chain(s2s, dsts, S2W)

    x_out = _tc_finalize(acc2, x, mask_ligand[:, None])
    return h_out, x_out
